# baseline (speedup 1.0000x reference)
import functools

import jax
import jax.numpy as jnp
from jax import lax
from jax.experimental import pallas as pl
from jax.experimental.pallas import tpu as pltpu
from jax.experimental.pallas import tpu_sc as plsc

N = 10000
D = 32
F = 32
EMB = 32
ENC = 64
HID = 32
LAYERS = 2
FLOW_ITERS = 8
DUAL_ITERS = 8
STEP = 0.01
MOM = 0.9
BIG = 1e9

NC = 2
NS = 16
NW = NC * NS
L = 16

NPAD = 10240
NT = NPAD // NW
ET = NT * D
CH = 256
NCH = ET // CH
NT_F = NPAD // NS
ET_F = NT_F * D

_MESH = plsc.VectorSubcoreMesh(
    core_axis_name="c", subcore_axis_name="s", num_cores=NC, num_subcores=NS)
_SC_PARAMS = pltpu.CompilerParams(
    needs_layout_passes=False, use_tc_tiling_on_sc=False)


def _enc_body(emb_ref, feat_ref, wenc_ref, benc_ref, wgat_ref, asrc_ref,
              adst_ref, h_ref, ssrc_ref, sdst_ref):
  emb = emb_ref[...]
  nrm = jnp.sqrt(jnp.sum(emb * emb, axis=-1, keepdims=True))
  emb = emb / jnp.maximum(nrm, 1.0)
  x = jnp.concatenate([emb, feat_ref[...]], axis=-1)
  st = jnp.dot(x, wenc_ref[...], preferred_element_type=jnp.float32)
  st = st + benc_ref[...]
  h = jnp.dot(st, wgat_ref[...], preferred_element_type=jnp.float32)
  h_ref[...] = h
  ssrc_ref[...] = jnp.dot(h, asrc_ref[...], preferred_element_type=jnp.float32)
  sdst_ref[...] = jnp.dot(h, adst_ref[...], preferred_element_type=jnp.float32)


def _gat_dense_body(agg_ref, wgat_ref, asrc_ref, adst_ref,
                    h_ref, ssrc_ref, sdst_ref):
  st = jnp.tanh(agg_ref[...])
  h = jnp.dot(st, wgat_ref[...], preferred_element_type=jnp.float32)
  h_ref[...] = h
  ssrc_ref[...] = jnp.dot(h, asrc_ref[...], preferred_element_type=jnp.float32)
  sdst_ref[...] = jnp.dot(h, adst_ref[...], preferred_element_type=jnp.float32)


def _dec_dense_body(agg_ref, w1a_ref, w1b_ref, wdu1_ref, bdu1_ref, wdu2_ref,
                    bdu2_ref, u_ref, w_ref, dv_ref):
  enc = jnp.tanh(agg_ref[...])
  u_ref[...] = jnp.dot(enc, w1a_ref[...], preferred_element_type=jnp.float32)
  w_ref[...] = jnp.dot(enc, w1b_ref[...], preferred_element_type=jnp.float32)
  hdu = jnp.tanh(
      jnp.dot(enc, wdu1_ref[...], preferred_element_type=jnp.float32)
      + bdu1_ref[...])
  dv_ref[...] = (jnp.dot(hdu, wdu2_ref[...], preferred_element_type=jnp.float32)
                 + bdu2_ref[...])


def _final_body(el_ref, dd_ref, adj_ref, flow_ref, dv_ref, dem_ref,
                dflow_ref, fc_ref, dc_ref, loss_ref):
  pid = pl.program_id(0)
  el = el_ref[...]
  dd = dd_ref[...]
  am = 1.0 - (adj_ref[...] == N).astype(jnp.float32)
  x = jnp.zeros_like(el)
  v = jnp.zeros_like(el)
  for _ in range(DUAL_ITERS):
    g = 2.0 * el * x + dd
    v = MOM * v - STEP * g
    x = jnp.maximum(x + v, 0.0) * am
  dflow_ref[...] = x
  fl = flow_ref[...]
  fpart = jnp.sum(el * fl * fl)
  dpart = jnp.sum(el * x * x + dd * x) - jnp.sum(dv_ref[...] * dem_ref[...])

  @pl.when(pid == 0)
  def _():
    fc_ref[0, 0] = 0.0
    dc_ref[0, 0] = 0.0

  fc_ref[0, 0] += fpart
  dc_ref[0, 0] += dpart

  @pl.when(pid == pl.num_programs(0) - 1)
  def _():
    loss_ref[0, 0] = fc_ref[0, 0] - dc_ref[0, 0]


def _dyn_bcast(vec, k):
  idx = jnp.full((L,), k, jnp.int32)
  return lax.gather(
      vec, idx[:, None],
      lax.GatherDimensionNumbers(offset_dims=(), collapsed_slice_dims=(0,),
                                 start_index_map=(0,)),
      slice_sizes=(1,),
      mode=lax.GatherScatterMode.PROMISE_IN_BOUNDS)


def _sc_tanh(x):
  ex = jnp.exp(2.0 * x)
  return (ex - 1.0) / (ex + 1.0)


def _gat_sc_body(h_hbm, ssrc_hbm, sdst_hbm, adj_hbm, agg_hbm,
                 sdst_v, ssrc_v, adj_v, alpha_v, agg_v, rows0, rows1,
                 sem0, sem1):
  cidx = lax.axis_index("c")
  sidx = lax.axis_index("s")
  wid = sidx * NC + cidx
  nb = wid * NT
  eb = wid * ET
  pltpu.sync_copy(sdst_hbm, sdst_v)
  pltpu.sync_copy(ssrc_hbm.at[pl.ds(nb, NT)], ssrc_v)
  pltpu.sync_copy(adj_hbm.at[pl.ds(eb, ET)], adj_v)

  def alpha_node(j, _):
    base = j * D
    es = []
    nms = []
    for g in range(2):
      idx = adj_v[pl.ds(base + g * L, L)]
      msk = idx == N
      adj_v[pl.ds(base + g * L, L)] = jnp.where(msk, 0, idx)
      sg = plsc.load_gather(sdst_v, [jnp.where(msk, 0, idx)])
      src = plsc.load_gather(ssrc_v, [jnp.full((L,), j, jnp.int32)])
      e = src + sg
      e = jnp.where(e >= 0.0, e, 0.2 * e)
      e = jnp.where(msk, -BIG, e)
      es.append(e)
      nms.append(1.0 - msk.astype(jnp.float32))
    mb = jnp.full((L,), jnp.max(jnp.maximum(es[0], es[1])), jnp.float32)
    p0 = jnp.exp(es[0] - mb) * nms[0]
    p1 = jnp.exp(es[1] - mb) * nms[1]
    sb = jnp.full((L,), jnp.sum(p0 + p1), jnp.float32)
    r = 1.0 / jnp.maximum(sb, 1e-30)
    alpha_v[pl.ds(base, L)] = p0 * r
    alpha_v[pl.ds(base + L, L)] = p1 * r
    return 0

  lax.fori_loop(0, NT, alpha_node, 0)

  def start(ci, rows, sem):
    pltpu.async_copy(h_hbm.at[adj_v.at[pl.ds(ci * CH, CH)]], rows, sem)

  def wait(ci, rows, sem):
    pltpu.make_async_copy(
        h_hbm.at[adj_v.at[pl.ds(ci * CH, CH)]], rows, sem).wait()

  def process(ci, rows):
    for j in range(CH // D):
      a0 = alpha_v[pl.ds(ci * CH + j * D, L)]
      a1 = alpha_v[pl.ds(ci * CH + j * D + L, L)]
      acc = [jnp.zeros((L,), jnp.float32) for _ in range(ENC // L)]
      for dd_ in range(D):
        e_loc = j * D + dd_
        a = _dyn_bcast(a0 if dd_ < L else a1, dd_ % L)
        for f in range(ENC // L):
          acc[f] = acc[f] + a * rows[e_loc, pl.ds(f * L, L)]
      nl = ci * (CH // D) + j
      for f in range(ENC // L):
        agg_v[pl.ds(nl * ENC + f * L, L)] = acc[f]

  start(0, rows0, sem0)

  def pipe(t, _):
    c0 = 2 * t
    c1 = 2 * t + 1
    start(c1, rows1, sem1)
    wait(c0, rows0, sem0)
    process(c0, rows0)

    @pl.when(c1 + 1 < NCH)
    def _():
      start(c1 + 1, rows0, sem0)

    wait(c1, rows1, sem1)
    process(c1, rows1)
    return 0

  lax.fori_loop(0, NCH // 2, pipe, 0)
  pltpu.sync_copy(agg_v, agg_hbm.at[pl.ds(nb * ENC, NT * ENC)])


def _dec_sc_body(u_hbm, w_hbm, dv_hbm, adj_hbm, b1_hbm, w2_hbm, b2_hbm,
                 pred_hbm, norm_hbm, ddf_hbm,
                 u_v, dv_v, adj_v, nm_v, b1_v, w2_v, b2_v,
                 pred_v, norm_v, dd_v, rows0, rows1, sem0, sem1):
  cidx = lax.axis_index("c")
  sidx = lax.axis_index("s")
  wid = sidx * NC + cidx
  nb = wid * NT
  eb = wid * ET
  pltpu.sync_copy(u_hbm.at[pl.ds(nb * HID, NT * HID)], u_v)
  pltpu.sync_copy(dv_hbm, dv_v)
  pltpu.sync_copy(adj_hbm.at[pl.ds(eb, ET)], adj_v)
  pltpu.sync_copy(b1_hbm, b1_v)
  pltpu.sync_copy(w2_hbm, w2_v)
  pltpu.sync_copy(b2_hbm, b2_v)

  def prep(g, _):
    idx = adj_v[pl.ds(g * L, L)]
    msk = idx == N
    adj_v[pl.ds(g * L, L)] = jnp.where(msk, 0, idx)
    nm_v[pl.ds(g * L, L)] = 1.0 - msk.astype(jnp.float32)
    return 0

  lax.fori_loop(0, ET // L, prep, 0)

  iota = lax.iota(jnp.int32, L)
  w2_r = [w2_v[pl.ds(0, L)], w2_v[pl.ds(L, L)]]
  b1_r = [b1_v[pl.ds(0, L)], b1_v[pl.ds(L, L)]]
  b2s = plsc.load_gather(b2_v, [jnp.zeros((L,), jnp.int32)])
  cms = jnp.full(
      (L,),
      jnp.sum(w2_r[0] * _sc_tanh(b1_r[0]) + w2_r[1] * _sc_tanh(b1_r[1])),
      jnp.float32) + b2s

  def start(ci, rows, sem):
    pltpu.async_copy(w_hbm.at[adj_v.at[pl.ds(ci * CH, CH)]], rows, sem)

  def wait(ci, rows, sem):
    pltpu.make_async_copy(
        w_hbm.at[adj_v.at[pl.ds(ci * CH, CH)]], rows, sem).wait()

  def process(ci, rows):
    for j in range(CH // D):
      nl = ci * (CH // D) + j
      pws = []
      nmvs = []
      for g in range(2):
        ebase = j * D + g * L
        e_glob = ci * CH + ebase
        nmv = nm_v[pl.ds(e_glob, L)]
        row_idx = iota + ebase
        nw = jnp.zeros((L,), jnp.float32)
        for half in range(2):
          u_r = u_v[pl.ds(nl * HID + half * L, L)]
          w2h = w2_r[half]

          def kb(k, acc, u_r=u_r, w2h=w2h, row_idx=row_idx, half=half):
            wk = plsc.load_gather(
                rows, [row_idx, jnp.full((L,), half * L + k, jnp.int32)])
            t = _sc_tanh(_dyn_bcast(u_r, k) + wk)
            return acc + _dyn_bcast(w2h, k) * t

          nw = lax.fori_loop(0, L, kb, nw, unroll=8)
        pw = jnp.where(nmv > 0.5, nw + b2s, cms)
        pw = pw - BIG * (1.0 - nmv)
        pred_v[pl.ds(e_glob, L)] = pw
        pws.append(pw)
        nmvs.append(nmv)
        saf = adj_v[pl.ds(e_glob, L)]
        dvg = plsc.load_gather(dv_v, [saf])
        dvn = plsc.load_gather(dv_v, [jnp.full((L,), nb + nl, jnp.int32)])
        dd_v[pl.ds(e_glob, L)] = nmv * (dvg - dvn)
      mb = jnp.full((L,), jnp.max(jnp.maximum(pws[0], pws[1])), jnp.float32)
      p0 = jnp.exp(pws[0] - mb)
      p1 = jnp.exp(pws[1] - mb)
      r = 1.0 / jnp.full((L,), jnp.sum(p0 + p1), jnp.float32)
      e0 = ci * CH + j * D
      norm_v[pl.ds(e0, L)] = p0 * r
      norm_v[pl.ds(e0 + L, L)] = p1 * r

  start(0, rows0, sem0)

  def pipe(t, _):
    c0 = 2 * t
    c1 = 2 * t + 1
    start(c1, rows1, sem1)
    wait(c0, rows0, sem0)
    process(c0, rows0)

    @pl.when(c1 + 1 < NCH)
    def _():
      start(c1 + 1, rows0, sem0)

    wait(c1, rows1, sem1)
    process(c1, rows1)
    return 0

  lax.fori_loop(0, NCH // 2, pipe, 0)
  pltpu.sync_copy(pred_v, pred_hbm.at[pl.ds(eb, ET)])
  pltpu.sync_copy(norm_v, norm_hbm.at[pl.ds(eb, ET)])
  pltpu.sync_copy(dd_v, ddf_hbm.at[pl.ds(eb, ET)])


def _flow_sc_body(norm_hbm, inidx_hbm, dem_hbm, flow_hbm,
                  norm_v, inidx_v, dem_v, infl_v, flow_v, sem):
  cidx = lax.axis_index("c")
  sidx = lax.axis_index("s")

  @pl.when(cidx == 0)
  def _():
    tb_n = sidx * NT_F
    tb_e = sidx * ET_F
    pltpu.sync_copy(norm_hbm.at[pl.ds(tb_e, ET_F)], norm_v)
    pltpu.sync_copy(inidx_hbm.at[pl.ds(tb_e, ET_F)], inidx_v)
    pltpu.sync_copy(dem_hbm.at[pl.ds(tb_n, NT_F)], dem_v)

    zero16 = jnp.zeros((L,), jnp.float32)

    def zinit(i, _):
      flow_v[pl.ds(i * L, L)] = zero16
      return 0

    lax.fori_loop(0, ET_F // L, zinit, 0)
    pltpu.sync_copy(flow_v, flow_hbm.at[pl.ds(tb_e, ET_F)])
    plsc.subcore_barrier()

    def one_iter(it, _):
      pltpu.async_copy(flow_hbm.at[inidx_v], infl_v, sem).wait()
      plsc.subcore_barrier()

      def node(j, _):
        base = j * D
        i0 = infl_v[pl.ds(base, L)]
        i1 = infl_v[pl.ds(base + L, L)]
        dem = plsc.load_gather(dem_v, [jnp.full((L,), j, jnp.int32)])
        sb = jnp.full((L,), jnp.sum(i0 + i1), jnp.float32)
        tot = jnp.maximum(dem + sb, 0.0)
        flow_v[pl.ds(base, L)] = norm_v[pl.ds(base, L)] * tot
        flow_v[pl.ds(base + L, L)] = norm_v[pl.ds(base + L, L)] * tot
        return 0

      lax.fori_loop(0, NT_F, node, 0)
      pltpu.sync_copy(flow_v, flow_hbm.at[pl.ds(tb_e, ET_F)])
      plsc.subcore_barrier()
      return 0

    lax.fori_loop(0, FLOW_ITERS, one_iter, 0)


_BLK = 1024


def _tc_enc(emb_p, feat_p, wenc, benc, wgat, asrc, adst):
  grid = (NPAD // _BLK,)
  full = lambda a: pl.BlockSpec(a.shape, lambda i: (0,) * a.ndim)
  return pl.pallas_call(
      _enc_body,
      grid=grid,
      in_specs=[
          pl.BlockSpec((_BLK, EMB), lambda i: (i, 0)),
          pl.BlockSpec((_BLK, F), lambda i: (i, 0)),
          full(wenc), full(benc), full(wgat), full(asrc), full(adst),
      ],
      out_specs=[
          pl.BlockSpec((_BLK, ENC), lambda i: (i, 0)),
          pl.BlockSpec((_BLK, 1), lambda i: (i, 0)),
          pl.BlockSpec((_BLK, 1), lambda i: (i, 0)),
      ],
      out_shape=[
          jax.ShapeDtypeStruct((NPAD, ENC), jnp.float32),
          jax.ShapeDtypeStruct((NPAD, 1), jnp.float32),
          jax.ShapeDtypeStruct((NPAD, 1), jnp.float32),
      ],
  )(emb_p, feat_p, wenc, benc, wgat, asrc, adst)


def _tc_gat_dense(agg, wgat, asrc, adst):
  grid = (NPAD // _BLK,)
  full = lambda a: pl.BlockSpec(a.shape, lambda i: (0,) * a.ndim)
  return pl.pallas_call(
      _gat_dense_body,
      grid=grid,
      in_specs=[
          pl.BlockSpec((_BLK, ENC), lambda i: (i, 0)),
          full(wgat), full(asrc), full(adst),
      ],
      out_specs=[
          pl.BlockSpec((_BLK, ENC), lambda i: (i, 0)),
          pl.BlockSpec((_BLK, 1), lambda i: (i, 0)),
          pl.BlockSpec((_BLK, 1), lambda i: (i, 0)),
      ],
      out_shape=[
          jax.ShapeDtypeStruct((NPAD, ENC), jnp.float32),
          jax.ShapeDtypeStruct((NPAD, 1), jnp.float32),
          jax.ShapeDtypeStruct((NPAD, 1), jnp.float32),
      ],
  )(agg, wgat, asrc, adst)


def _tc_dec_dense(agg, w1a, w1b, wdu1, bdu1, wdu2, bdu2):
  grid = (NPAD // _BLK,)
  full = lambda a: pl.BlockSpec(a.shape, lambda i: (0,) * a.ndim)
  return pl.pallas_call(
      _dec_dense_body,
      grid=grid,
      in_specs=[
          pl.BlockSpec((_BLK, ENC), lambda i: (i, 0)),
          full(w1a), full(w1b), full(wdu1), full(bdu1), full(wdu2), full(bdu2),
      ],
      out_specs=[
          pl.BlockSpec((_BLK, HID), lambda i: (i, 0)),
          pl.BlockSpec((_BLK, HID), lambda i: (i, 0)),
          pl.BlockSpec((_BLK, 1), lambda i: (i, 0)),
      ],
      out_shape=[
          jax.ShapeDtypeStruct((NPAD, HID), jnp.float32),
          jax.ShapeDtypeStruct((NPAD, HID), jnp.float32),
          jax.ShapeDtypeStruct((NPAD, 1), jnp.float32),
      ],
  )(agg, w1a, w1b, wdu1, bdu1, wdu2, bdu2)


_FBLK = 1000


def _tc_final(el, ddm, adj, flow, dv, dem):
  grid = (N // _FBLK,)
  one = lambda: pl.BlockSpec((1, 1), lambda i: (0, 0),
                             memory_space=pltpu.SMEM)
  return pl.pallas_call(
      _final_body,
      grid=grid,
      in_specs=[
          pl.BlockSpec((_FBLK, D), lambda i: (i, 0)),
          pl.BlockSpec((_FBLK, D), lambda i: (i, 0)),
          pl.BlockSpec((_FBLK, D), lambda i: (i, 0)),
          pl.BlockSpec((_FBLK, D), lambda i: (i, 0)),
          pl.BlockSpec((_FBLK, 1), lambda i: (i, 0)),
          pl.BlockSpec((_FBLK, 1), lambda i: (i, 0)),
      ],
      out_specs=[pl.BlockSpec((_FBLK, D), lambda i: (i, 0)), one(), one(),
                 one()],
      out_shape=[
          jax.ShapeDtypeStruct((N, D), jnp.float32),
          jax.ShapeDtypeStruct((1, 1), jnp.float32),
          jax.ShapeDtypeStruct((1, 1), jnp.float32),
          jax.ShapeDtypeStruct((1, 1), jnp.float32),
      ],
  )(el, ddm, adj, flow, dv, dem)


_gat_sc = functools.partial(
    pl.kernel,
    out_type=[jax.ShapeDtypeStruct((NPAD * ENC,), jnp.float32)],
    mesh=_MESH,
    scratch_types=[
        pltpu.VMEM((NPAD,), jnp.float32),
        pltpu.VMEM((NT,), jnp.float32),
        pltpu.VMEM((ET,), jnp.int32),
        pltpu.VMEM((ET,), jnp.float32),
        pltpu.VMEM((NT * ENC,), jnp.float32),
        pltpu.VMEM((CH, ENC), jnp.float32),
        pltpu.VMEM((CH, ENC), jnp.float32),
        pltpu.SemaphoreType.DMA,
        pltpu.SemaphoreType.DMA,
    ],
    compiler_params=_SC_PARAMS,
)(_gat_sc_body)


_dec_sc = functools.partial(
    pl.kernel,
    out_type=[
        jax.ShapeDtypeStruct((NPAD * D,), jnp.float32),
        jax.ShapeDtypeStruct((NPAD * D,), jnp.float32),
        jax.ShapeDtypeStruct((NPAD * D,), jnp.float32),
    ],
    mesh=_MESH,
    scratch_types=[
        pltpu.VMEM((NT * HID,), jnp.float32),
        pltpu.VMEM((NPAD,), jnp.float32),
        pltpu.VMEM((ET,), jnp.int32),
        pltpu.VMEM((ET,), jnp.float32),
        pltpu.VMEM((HID,), jnp.float32),
        pltpu.VMEM((HID,), jnp.float32),
        pltpu.VMEM((16,), jnp.float32),
        pltpu.VMEM((ET,), jnp.float32),
        pltpu.VMEM((ET,), jnp.float32),
        pltpu.VMEM((ET,), jnp.float32),
        pltpu.VMEM((CH, HID), jnp.float32),
        pltpu.VMEM((CH, HID), jnp.float32),
        pltpu.SemaphoreType.DMA,
        pltpu.SemaphoreType.DMA,
    ],
    compiler_params=_SC_PARAMS,
)(_dec_sc_body)


_flow_sc = functools.partial(
    pl.kernel,
    out_type=[jax.ShapeDtypeStruct((NPAD * D,), jnp.float32)],
    mesh=_MESH,
    scratch_types=[
        pltpu.VMEM((ET_F,), jnp.float32),
        pltpu.VMEM((ET_F,), jnp.int32),
        pltpu.VMEM((NT_F,), jnp.float32),
        pltpu.VMEM((ET_F,), jnp.float32),
        pltpu.VMEM((ET_F,), jnp.float32),
        pltpu.SemaphoreType.DMA,
    ],
    compiler_params=_SC_PARAMS,
)(_flow_sc_body)


def kernel(demands, node_features, adj_lst, inv_adj_lst, edge_lengths,
           norm_edge_lengths, common_neighbors, neighborhoods, in_indices,
           rev_indices, num_nodes, emb_table, W_enc, b_enc, W_gat, a_src,
           a_dst, W_dec1, b_dec1, W_dec2, b_dec2, W_du1, b_du1, W_du2, b_du2):
  del inv_adj_lst, norm_edge_lengths, common_neighbors, neighborhoods
  del rev_indices, num_nodes
  pad_n = NPAD - N

  dem = demands[0, :, 0]
  feat = node_features[0]
  adj = adj_lst[0]
  el = edge_lengths[0]
  in_idx = in_indices[0]

  emb_p = jnp.pad(emb_table, ((0, pad_n), (0, 0)))
  feat_p = jnp.pad(feat, ((0, pad_n), (0, 0)))
  adjf = jnp.pad(adj, ((0, pad_n), (0, 0)), constant_values=N).reshape(-1)
  inf_p = jnp.pad(in_idx, ((0, pad_n), (0, 0))).reshape(-1)
  dem_p = jnp.pad(dem, (0, pad_n))

  benc = b_enc.reshape(1, ENC)
  asrc = a_src.reshape(ENC, 1)
  adst = a_dst.reshape(ENC, 1)
  w1a = W_dec1[:ENC]
  w1b = W_dec1[ENC:]
  bdu1 = b_du1.reshape(1, HID)
  bdu2 = b_du2.reshape(1, 1)
  b2_p = jnp.pad(b_dec2, (0, 15))

  h, ssrc, sdst = _tc_enc(emb_p, feat_p, W_enc, benc, W_gat, asrc, adst)
  (agg1,) = _gat_sc(h, ssrc.reshape(-1), sdst.reshape(-1), adjf)
  h2, ssrc2, sdst2 = _tc_gat_dense(agg1.reshape(NPAD, ENC), W_gat, asrc, adst)
  (agg2,) = _gat_sc(h2, ssrc2.reshape(-1), sdst2.reshape(-1), adjf)
  u, w, dv = _tc_dec_dense(agg2.reshape(NPAD, ENC), w1a, w1b, W_du1, bdu1,
                           W_du2, bdu2)
  pred_f, norm_f, dd_f = _dec_sc(u.reshape(-1), w, dv.reshape(-1), adjf,
                                 b_dec1, W_dec2.reshape(-1), b2_p)
  (flow_f,) = _flow_sc(norm_f, inf_p, dem_p)

  flow2 = flow_f.reshape(NPAD, D)[:N]
  dd2 = dd_f.reshape(NPAD, D)[:N]
  dflow, fc, dc, loss = _tc_final(el, dd2, adj, flow2, dv[:N],
                                  demands[0])

  normalized = norm_f.reshape(NPAD, D)[:N][None]
  pred = pred_f.reshape(NPAD, D)[:N][None]
  return (flow2[None], fc.reshape(1), normalized, dc.reshape(1), pred,
          dflow[None], jnp.zeros((1,), jnp.int32), loss.reshape(1))


# speedup vs baseline: 1.4718x; 1.4718x over previous
import functools

import jax
import jax.numpy as jnp
from jax import lax
from jax.experimental import pallas as pl
from jax.experimental.pallas import tpu as pltpu
from jax.experimental.pallas import tpu_sc as plsc

N = 10000
D = 32
F = 32
EMB = 32
ENC = 64
HID = 32
LAYERS = 2
FLOW_ITERS = 8
DUAL_ITERS = 8
STEP = 0.01
MOM = 0.9
BIG = 1e9

NC = 2
NS = 16
NW = NC * NS
L = 16

NPAD = 10240
NT = NPAD // NW
ET = NT * D
NT0 = 448
NT1 = 192
ET0 = NT0 * D
ET1 = NT1 * D
CH = 256
NCH0 = ET0 // CH
NCH1 = ET1 // CH
CHD = 128
NCHD = ET // CHD
NT_F = NPAD // NS
ET_F = NT_F * D

_MESH = plsc.VectorSubcoreMesh(
    core_axis_name="c", subcore_axis_name="s", num_cores=NC, num_subcores=NS)
_SC_PARAMS = pltpu.CompilerParams(
    needs_layout_passes=False, use_tc_tiling_on_sc=False)


def _enc_body(emb_ref, feat_ref, wenc_ref, benc_ref, wgat_ref, asrc_ref,
              adst_ref, h_ref, ssrc_ref, sdst_ref):
  emb = emb_ref[...]
  nrm = jnp.sqrt(jnp.sum(emb * emb, axis=-1, keepdims=True))
  emb = emb / jnp.maximum(nrm, 1.0)
  x = jnp.concatenate([emb, feat_ref[...]], axis=-1)
  st = jnp.dot(x, wenc_ref[...], preferred_element_type=jnp.float32)
  st = st + benc_ref[...]
  h = jnp.dot(st, wgat_ref[...], preferred_element_type=jnp.float32)
  h_ref[...] = h
  ssrc_ref[...] = jnp.dot(h, asrc_ref[...], preferred_element_type=jnp.float32)
  sdst_ref[...] = jnp.dot(h, adst_ref[...], preferred_element_type=jnp.float32)


def _gat_dense_body(agg_ref, wgat_ref, asrc_ref, adst_ref,
                    h_ref, ssrc_ref, sdst_ref):
  st = jnp.tanh(agg_ref[...])
  h = jnp.dot(st, wgat_ref[...], preferred_element_type=jnp.float32)
  h_ref[...] = h
  ssrc_ref[...] = jnp.dot(h, asrc_ref[...], preferred_element_type=jnp.float32)
  sdst_ref[...] = jnp.dot(h, adst_ref[...], preferred_element_type=jnp.float32)


def _dec_dense_body(agg_ref, w1a_ref, w1b_ref, wdu1_ref, bdu1_ref, wdu2_ref,
                    bdu2_ref, u_ref, w_ref, dv_ref):
  enc = jnp.tanh(agg_ref[...])
  u_ref[...] = jnp.dot(enc, w1a_ref[...], preferred_element_type=jnp.float32)
  w_ref[...] = jnp.dot(enc, w1b_ref[...], preferred_element_type=jnp.float32)
  hdu = jnp.tanh(
      jnp.dot(enc, wdu1_ref[...], preferred_element_type=jnp.float32)
      + bdu1_ref[...])
  dv_ref[...] = (jnp.dot(hdu, wdu2_ref[...], preferred_element_type=jnp.float32)
                 + bdu2_ref[...])


def _final_body(el_ref, dd_ref, adj_ref, flow_ref, dv_ref, dem_ref,
                dflow_ref, fc_ref, dc_ref, loss_ref):
  pid = pl.program_id(0)
  el = el_ref[...]
  dd = dd_ref[...]
  am = 1.0 - (adj_ref[...] == N).astype(jnp.float32)
  x = jnp.zeros_like(el)
  v = jnp.zeros_like(el)
  for _ in range(DUAL_ITERS):
    g = 2.0 * el * x + dd
    v = MOM * v - STEP * g
    x = jnp.maximum(x + v, 0.0) * am
  dflow_ref[...] = x
  fl = flow_ref[...]
  fpart = jnp.sum(el * fl * fl)
  dpart = jnp.sum(el * x * x + dd * x) - jnp.sum(dv_ref[...] * dem_ref[...])

  @pl.when(pid == 0)
  def _():
    fc_ref[0, 0] = 0.0
    dc_ref[0, 0] = 0.0

  fc_ref[0, 0] += fpart
  dc_ref[0, 0] += dpart

  @pl.when(pid == pl.num_programs(0) - 1)
  def _():
    loss_ref[0, 0] = fc_ref[0, 0] - dc_ref[0, 0]


def _dyn_bcast(vec, k):
  idx = jnp.full((L,), k, jnp.int32)
  return lax.gather(
      vec, idx[:, None],
      lax.GatherDimensionNumbers(offset_dims=(), collapsed_slice_dims=(0,),
                                 start_index_map=(0,)),
      slice_sizes=(1,),
      mode=lax.GatherScatterMode.PROMISE_IN_BOUNDS)


def _sc_tanh(x):
  ex = jnp.exp(2.0 * x)
  return (ex - 1.0) / (ex + 1.0)


def _gat_sc_body(h_hbm, ssrc_hbm, sdst_hbm, adj_hbm, agg_hbm,
                 sdst_v, ssrc_v, adj_v, nm_v, alpha_v, agg_v, rows0, rows1,
                 sem0, sem1):
  cidx = lax.axis_index("c")
  sidx = lax.axis_index("s")
  is0 = cidx == 0
  n_t = jnp.where(is0, NT0, NT1)
  nb = jnp.where(is0, sidx * NT0, NT0 * NS + sidx * NT1)
  eb = nb * D
  nch = jnp.where(is0, NCH0, NCH1)
  pltpu.sync_copy(sdst_hbm, sdst_v)

  @pl.when(is0)
  def _():
    pltpu.sync_copy(ssrc_hbm.at[pl.ds(sidx * NT0, NT0)],
                    ssrc_v.at[pl.ds(0, NT0)])
    pltpu.sync_copy(adj_hbm.at[pl.ds(sidx * ET0, ET0)],
                    adj_v.at[pl.ds(0, ET0)])

  @pl.when(cidx == 1)
  def _():
    pltpu.sync_copy(ssrc_hbm.at[pl.ds(NT0 * NS + sidx * NT1, NT1)],
                    ssrc_v.at[pl.ds(0, NT1)])
    pltpu.sync_copy(adj_hbm.at[pl.ds(NT0 * NS * D + sidx * ET1, ET1)],
                    adj_v.at[pl.ds(0, ET1)])

  def prep(g, _):
    idx = adj_v[pl.ds(g * L, L)]
    adj_v[pl.ds(g * L, L)] = jnp.where(idx == N, 0, idx)
    nm_v[pl.ds(g * L, L)] = 1.0 - (idx == N).astype(jnp.float32)
    return 0

  lax.fori_loop(0, n_t * D // L, prep, 0)

  def alpha_node(j, _):
    base = j * D
    es = []
    nms = []
    for g in range(2):
      saf = adj_v[pl.ds(base + g * L, L)]
      nmk = nm_v[pl.ds(base + g * L, L)]
      msk = nmk < 0.5
      sg = plsc.load_gather(sdst_v, [saf])
      src = plsc.load_gather(ssrc_v, [jnp.full((L,), j, jnp.int32)])
      e = src + sg
      e = jnp.where(e >= 0.0, e, 0.2 * e)
      e = jnp.where(msk, -BIG, e)
      es.append(e)
      nms.append(nmk)
    mb = jnp.full((L,), jnp.max(jnp.maximum(es[0], es[1])), jnp.float32)
    p0 = jnp.exp(es[0] - mb) * nms[0]
    p1 = jnp.exp(es[1] - mb) * nms[1]
    sb = jnp.full((L,), jnp.sum(p0 + p1), jnp.float32)
    r = 1.0 / jnp.maximum(sb, 1e-30)
    alpha_v[pl.ds(base, L)] = p0 * r
    alpha_v[pl.ds(base + L, L)] = p1 * r
    return 0

  lax.fori_loop(0, n_t, alpha_node, 0)

  def start(ci, rows, sem):
    pltpu.async_copy(h_hbm.at[adj_v.at[pl.ds(ci * CH, CH)]], rows, sem)

  def wait(ci, rows, sem):
    pltpu.make_async_copy(
        h_hbm.at[adj_v.at[pl.ds(ci * CH, CH)]], rows, sem).wait()

  _PROCESS_ANCHOR = 0

  def process(ci, rows):
    for j in range(CH // D):
      a0 = alpha_v[pl.ds(ci * CH + j * D, L)]
      a1 = alpha_v[pl.ds(ci * CH + j * D + L, L)]
      acc = [jnp.zeros((L,), jnp.float32) for _ in range(ENC // L)]
      for dd_ in range(D):
        e_loc = j * D + dd_
        a = _dyn_bcast(a0 if dd_ < L else a1, dd_ % L)
        for f in range(ENC // L):
          acc[f] = acc[f] + a * rows[e_loc, pl.ds(f * L, L)]
      nl = ci * (CH // D) + j
      for f in range(ENC // L):
        agg_v[pl.ds(nl * ENC + f * L, L)] = acc[f]

  start(0, rows0, sem0)
  start(1, rows1, sem1)
  lax.fori_loop(0, n_t, alpha_node, 0)

  def pipe(t, _):
    c0 = 2 * t
    c1 = 2 * t + 1
    wait(c0, rows0, sem0)
    process(c0, rows0)

    @pl.when(c1 + 1 < nch)
    def _():
      start(c1 + 1, rows0, sem0)

    wait(c1, rows1, sem1)
    process(c1, rows1)

    @pl.when(c1 + 2 < nch)
    def _():
      start(c1 + 2, rows1, sem1)

    return 0

  lax.fori_loop(0, nch // 2, pipe, 0)

  @pl.when(is0)
  def _():
    pltpu.sync_copy(agg_v.at[pl.ds(0, NT0 * ENC)],
                    agg_hbm.at[pl.ds(sidx * NT0 * ENC, NT0 * ENC)])

  @pl.when(cidx == 1)
  def _():
    pltpu.sync_copy(agg_v.at[pl.ds(0, NT1 * ENC)],
                    agg_hbm.at[pl.ds((NT0 * NS + sidx * NT1) * ENC,
                                     NT1 * ENC)])


def _dec_sc_body(u_hbm, w_hbm, dv_hbm, adj_hbm, b1_hbm, w2_hbm, b2_hbm,
                 pred_hbm, norm_hbm, ddf_hbm,
                 u_v, dv_v, adj_v, nm_v, b1_v, w2_v, b2_v,
                 pred_v, norm_v, dd_v, rows0, rows1, sem0, sem1):
  cidx = lax.axis_index("c")
  sidx = lax.axis_index("s")
  wid = sidx * NC + cidx
  nb = wid * NT
  eb = wid * ET
  pltpu.sync_copy(u_hbm.at[pl.ds(nb * HID, NT * HID)], u_v)
  pltpu.sync_copy(dv_hbm, dv_v)
  pltpu.sync_copy(adj_hbm.at[pl.ds(eb, ET)], adj_v)
  pltpu.sync_copy(b1_hbm, b1_v)
  pltpu.sync_copy(w2_hbm, w2_v)
  pltpu.sync_copy(b2_hbm, b2_v)

  def prep(g, _):
    idx = adj_v[pl.ds(g * L, L)]
    msk = idx == N
    adj_v[pl.ds(g * L, L)] = jnp.where(msk, 0, idx)
    nm_v[pl.ds(g * L, L)] = 1.0 - msk.astype(jnp.float32)
    return 0

  lax.fori_loop(0, ET // L, prep, 0)

  iota = lax.iota(jnp.int32, L)

  def start(ci, rows, sem):
    pltpu.async_copy(w_hbm.at[adj_v.at[pl.ds(ci * CHD, CHD)]], rows, sem)

  def wait(ci, rows, sem):
    pltpu.make_async_copy(
        w_hbm.at[adj_v.at[pl.ds(ci * CHD, CHD)]], rows, sem).wait()

  def process(ci, rows):
    for j in range(CHD // D):
      nl = ci * (CHD // D) + j
      pws = []
      for g in range(2):
        ebase = j * D + g * L
        e_glob = ci * CHD + ebase
        nmv = nm_v[pl.ds(e_glob, L)]
        row_idx = iota + ebase
        nw = jnp.zeros((L,), jnp.float32)
        for half in range(2):
          u_r = u_v[pl.ds(nl * HID + half * L, L)]
          b1_r = b1_v[pl.ds(half * L, L)]
          w2_r = w2_v[pl.ds(half * L, L)]

          def kbody(k, nw, row_idx=row_idx, nmv=nmv, u_r=u_r, b1_r=b1_r,
                    w2_r=w2_r, half=half):
            wk = plsc.load_gather(
                rows, [row_idx, jnp.full((L,), half * L, jnp.int32) + k])
            pre = nmv * (_dyn_bcast(u_r, k) + wk) + _dyn_bcast(b1_r, k)
            ex = jnp.exp(2.0 * pre)
            t = (ex - 1.0) / (ex + 1.0)
            return nw + _dyn_bcast(w2_r, k) * t

          nw = lax.fori_loop(0, L, kbody, nw)
        nw = nw + plsc.load_gather(b2_v, [jnp.zeros((L,), jnp.int32)])
        pw = nw - BIG * (1.0 - nmv)
        pred_v[pl.ds(e_glob, L)] = pw
        pws.append(pw)
        saf = adj_v[pl.ds(e_glob, L)]
        dvg = plsc.load_gather(dv_v, [saf])
        dvn = plsc.load_gather(dv_v, [jnp.full((L,), nb + nl, jnp.int32)])
        dd_v[pl.ds(e_glob, L)] = nmv * (dvg - dvn)
      mb = jnp.full((L,), jnp.max(jnp.maximum(pws[0], pws[1])), jnp.float32)
      p0 = jnp.exp(pws[0] - mb)
      p1 = jnp.exp(pws[1] - mb)
      r = 1.0 / jnp.full((L,), jnp.sum(p0 + p1), jnp.float32)
      e0 = ci * CHD + j * D
      norm_v[pl.ds(e0, L)] = p0 * r
      norm_v[pl.ds(e0 + L, L)] = p1 * r

  start(0, rows0, sem0)

  def pipe(t, _):
    c0 = 2 * t
    c1 = 2 * t + 1
    start(c1, rows1, sem1)
    wait(c0, rows0, sem0)
    process(c0, rows0)

    @pl.when(c1 + 1 < NCHD)
    def _():
      start(c1 + 1, rows0, sem0)

    wait(c1, rows1, sem1)
    process(c1, rows1)
    return 0

  lax.fori_loop(0, NCHD // 2, pipe, 0)
  pltpu.sync_copy(pred_v, pred_hbm.at[pl.ds(eb, ET)])
  pltpu.sync_copy(norm_v, norm_hbm.at[pl.ds(eb, ET)])
  pltpu.sync_copy(dd_v, ddf_hbm.at[pl.ds(eb, ET)])


def _flow_sc_body(norm_hbm, inidx_hbm, dem_hbm, flow_hbm,
                  flow_sh, norm_v, inidx_v, dem_v, infl_v, flow_v, sem):
  cidx = lax.axis_index("c")
  sidx = lax.axis_index("s")

  @pl.when(cidx == 0)
  def _():
    tb_n = sidx * NT_F
    tb_e = sidx * ET_F
    pltpu.sync_copy(norm_hbm.at[pl.ds(tb_e, ET_F)], norm_v)
    pltpu.sync_copy(inidx_hbm.at[pl.ds(tb_e, ET_F)], inidx_v)
    pltpu.sync_copy(dem_hbm.at[pl.ds(tb_n, NT_F)], dem_v)

    zero16 = jnp.zeros((L,), jnp.float32)

    def zinit(i, _):
      flow_v[pl.ds(i * L, L)] = zero16
      return 0

    lax.fori_loop(0, ET_F // L, zinit, 0)
    pltpu.sync_copy(flow_v, flow_sh.at[pl.ds(tb_e, ET_F)])
    plsc.subcore_barrier()

    def one_iter(it, _):
      pltpu.async_copy(flow_sh.at[inidx_v], infl_v, sem).wait()
      plsc.subcore_barrier()

      def node(j, _):
        base = j * D
        i0 = infl_v[pl.ds(base, L)]
        i1 = infl_v[pl.ds(base + L, L)]
        dem = plsc.load_gather(dem_v, [jnp.full((L,), j, jnp.int32)])
        sb = jnp.full((L,), jnp.sum(i0 + i1), jnp.float32)
        tot = jnp.maximum(dem + sb, 0.0)
        flow_v[pl.ds(base, L)] = norm_v[pl.ds(base, L)] * tot
        flow_v[pl.ds(base + L, L)] = norm_v[pl.ds(base + L, L)] * tot
        return 0

      lax.fori_loop(0, NT_F, node, 0)
      pltpu.sync_copy(flow_v, flow_sh.at[pl.ds(tb_e, ET_F)])
      plsc.subcore_barrier()
      return 0

    lax.fori_loop(0, FLOW_ITERS, one_iter, 0)
    pltpu.sync_copy(flow_v, flow_hbm.at[pl.ds(tb_e, ET_F)])


_BLK = 1024


def _tc_enc(emb_p, feat_p, wenc, benc, wgat, asrc, adst):
  grid = (NPAD // _BLK,)
  full = lambda a: pl.BlockSpec(a.shape, lambda i: (0,) * a.ndim)
  return pl.pallas_call(
      _enc_body,
      grid=grid,
      in_specs=[
          pl.BlockSpec((_BLK, EMB), lambda i: (i, 0)),
          pl.BlockSpec((_BLK, F), lambda i: (i, 0)),
          full(wenc), full(benc), full(wgat), full(asrc), full(adst),
      ],
      out_specs=[
          pl.BlockSpec((_BLK, ENC), lambda i: (i, 0)),
          pl.BlockSpec((_BLK, 1), lambda i: (i, 0)),
          pl.BlockSpec((_BLK, 1), lambda i: (i, 0)),
      ],
      out_shape=[
          jax.ShapeDtypeStruct((NPAD, ENC), jnp.float32),
          jax.ShapeDtypeStruct((NPAD, 1), jnp.float32),
          jax.ShapeDtypeStruct((NPAD, 1), jnp.float32),
      ],
  )(emb_p, feat_p, wenc, benc, wgat, asrc, adst)


def _tc_gat_dense(agg, wgat, asrc, adst):
  grid = (NPAD // _BLK,)
  full = lambda a: pl.BlockSpec(a.shape, lambda i: (0,) * a.ndim)
  return pl.pallas_call(
      _gat_dense_body,
      grid=grid,
      in_specs=[
          pl.BlockSpec((_BLK, ENC), lambda i: (i, 0)),
          full(wgat), full(asrc), full(adst),
      ],
      out_specs=[
          pl.BlockSpec((_BLK, ENC), lambda i: (i, 0)),
          pl.BlockSpec((_BLK, 1), lambda i: (i, 0)),
          pl.BlockSpec((_BLK, 1), lambda i: (i, 0)),
      ],
      out_shape=[
          jax.ShapeDtypeStruct((NPAD, ENC), jnp.float32),
          jax.ShapeDtypeStruct((NPAD, 1), jnp.float32),
          jax.ShapeDtypeStruct((NPAD, 1), jnp.float32),
      ],
  )(agg, wgat, asrc, adst)


def _tc_dec_dense(agg, w1a, w1b, wdu1, bdu1, wdu2, bdu2):
  grid = (NPAD // _BLK,)
  full = lambda a: pl.BlockSpec(a.shape, lambda i: (0,) * a.ndim)
  return pl.pallas_call(
      _dec_dense_body,
      grid=grid,
      in_specs=[
          pl.BlockSpec((_BLK, ENC), lambda i: (i, 0)),
          full(w1a), full(w1b), full(wdu1), full(bdu1), full(wdu2), full(bdu2),
      ],
      out_specs=[
          pl.BlockSpec((_BLK, HID), lambda i: (i, 0)),
          pl.BlockSpec((_BLK, HID), lambda i: (i, 0)),
          pl.BlockSpec((_BLK, 1), lambda i: (i, 0)),
      ],
      out_shape=[
          jax.ShapeDtypeStruct((NPAD, HID), jnp.float32),
          jax.ShapeDtypeStruct((NPAD, HID), jnp.float32),
          jax.ShapeDtypeStruct((NPAD, 1), jnp.float32),
      ],
  )(agg, w1a, w1b, wdu1, bdu1, wdu2, bdu2)


_FBLK = 1000


def _tc_final(el, ddm, adj, flow, dv, dem):
  grid = (N // _FBLK,)
  one = lambda: pl.BlockSpec((1, 1), lambda i: (0, 0),
                             memory_space=pltpu.SMEM)
  return pl.pallas_call(
      _final_body,
      grid=grid,
      in_specs=[
          pl.BlockSpec((_FBLK, D), lambda i: (i, 0)),
          pl.BlockSpec((_FBLK, D), lambda i: (i, 0)),
          pl.BlockSpec((_FBLK, D), lambda i: (i, 0)),
          pl.BlockSpec((_FBLK, D), lambda i: (i, 0)),
          pl.BlockSpec((_FBLK, 1), lambda i: (i, 0)),
          pl.BlockSpec((_FBLK, 1), lambda i: (i, 0)),
      ],
      out_specs=[pl.BlockSpec((_FBLK, D), lambda i: (i, 0)), one(), one(),
                 one()],
      out_shape=[
          jax.ShapeDtypeStruct((N, D), jnp.float32),
          jax.ShapeDtypeStruct((1, 1), jnp.float32),
          jax.ShapeDtypeStruct((1, 1), jnp.float32),
          jax.ShapeDtypeStruct((1, 1), jnp.float32),
      ],
  )(el, ddm, adj, flow, dv, dem)


_gat_sc = functools.partial(
    pl.kernel,
    out_type=[jax.ShapeDtypeStruct((NPAD * ENC,), jnp.float32)],
    mesh=_MESH,
    scratch_types=[
        pltpu.VMEM((NPAD,), jnp.float32),
        pltpu.VMEM((NT0,), jnp.float32),
        pltpu.VMEM((ET0,), jnp.int32),
        pltpu.VMEM((ET0,), jnp.float32),
        pltpu.VMEM((ET0,), jnp.float32),
        pltpu.VMEM((NT0 * ENC,), jnp.float32),
        pltpu.VMEM((CH, ENC), jnp.float32),
        pltpu.VMEM((CH, ENC), jnp.float32),
        pltpu.SemaphoreType.DMA,
        pltpu.SemaphoreType.DMA,
    ],
    compiler_params=_SC_PARAMS,
)(_gat_sc_body)


_dec_sc = functools.partial(
    pl.kernel,
    out_type=[
        jax.ShapeDtypeStruct((NPAD * D,), jnp.float32),
        jax.ShapeDtypeStruct((NPAD * D,), jnp.float32),
        jax.ShapeDtypeStruct((NPAD * D,), jnp.float32),
    ],
    mesh=_MESH,
    scratch_types=[
        pltpu.VMEM((NT * HID,), jnp.float32),
        pltpu.VMEM((NPAD,), jnp.float32),
        pltpu.VMEM((ET,), jnp.int32),
        pltpu.VMEM((ET,), jnp.float32),
        pltpu.VMEM((HID,), jnp.float32),
        pltpu.VMEM((HID,), jnp.float32),
        pltpu.VMEM((16,), jnp.float32),
        pltpu.VMEM((ET,), jnp.float32),
        pltpu.VMEM((ET,), jnp.float32),
        pltpu.VMEM((ET,), jnp.float32),
        pltpu.VMEM((CHD, HID), jnp.float32),
        pltpu.VMEM((CHD, HID), jnp.float32),
        pltpu.SemaphoreType.DMA,
        pltpu.SemaphoreType.DMA,
    ],
    compiler_params=_SC_PARAMS,
)(_dec_sc_body)


_flow_sc = functools.partial(
    pl.kernel,
    out_type=[jax.ShapeDtypeStruct((NPAD * D,), jnp.float32)],
    mesh=_MESH,
    scratch_types=[
        pltpu.VMEM_SHARED((NPAD * D,), jnp.float32),
        pltpu.VMEM((ET_F,), jnp.float32),
        pltpu.VMEM((ET_F,), jnp.int32),
        pltpu.VMEM((NT_F,), jnp.float32),
        pltpu.VMEM((ET_F,), jnp.float32),
        pltpu.VMEM((ET_F,), jnp.float32),
        pltpu.SemaphoreType.DMA,
    ],
    compiler_params=_SC_PARAMS,
)(_flow_sc_body)


def kernel(demands, node_features, adj_lst, inv_adj_lst, edge_lengths,
           norm_edge_lengths, common_neighbors, neighborhoods, in_indices,
           rev_indices, num_nodes, emb_table, W_enc, b_enc, W_gat, a_src,
           a_dst, W_dec1, b_dec1, W_dec2, b_dec2, W_du1, b_du1, W_du2, b_du2):
  del inv_adj_lst, norm_edge_lengths, common_neighbors, neighborhoods
  del rev_indices, num_nodes
  pad_n = NPAD - N

  dem = demands[0, :, 0]
  feat = node_features[0]
  adj = adj_lst[0]
  el = edge_lengths[0]
  in_idx = in_indices[0]

  emb_p = jnp.pad(emb_table, ((0, pad_n), (0, 0)))
  feat_p = jnp.pad(feat, ((0, pad_n), (0, 0)))
  adjf = jnp.pad(adj, ((0, pad_n), (0, 0)), constant_values=N).reshape(-1)
  inf_p = jnp.pad(in_idx, ((0, pad_n), (0, 0))).reshape(-1)
  dem_p = jnp.pad(dem, (0, pad_n))

  benc = b_enc.reshape(1, ENC)
  asrc = a_src.reshape(ENC, 1)
  adst = a_dst.reshape(ENC, 1)
  w1a = W_dec1[:ENC]
  w1b = W_dec1[ENC:]
  bdu1 = b_du1.reshape(1, HID)
  bdu2 = b_du2.reshape(1, 1)
  b2_p = jnp.pad(b_dec2, (0, 15))

  h, ssrc, sdst = _tc_enc(emb_p, feat_p, W_enc, benc, W_gat, asrc, adst)
  (agg1,) = _gat_sc(h, ssrc.reshape(-1), sdst.reshape(-1), adjf)
  h2, ssrc2, sdst2 = _tc_gat_dense(agg1.reshape(NPAD, ENC), W_gat, asrc, adst)
  (agg2,) = _gat_sc(h2, ssrc2.reshape(-1), sdst2.reshape(-1), adjf)
  u, w, dv = _tc_dec_dense(agg2.reshape(NPAD, ENC), w1a, w1b, W_du1, bdu1,
                           W_du2, bdu2)
  pred_f, norm_f, dd_f = _dec_sc(u.reshape(-1), w, dv.reshape(-1), adjf,
                                 b_dec1, W_dec2.reshape(-1), b2_p)
  (flow_f,) = _flow_sc(norm_f, inf_p, dem_p)

  flow2 = flow_f.reshape(NPAD, D)[:N]
  dd2 = dd_f.reshape(NPAD, D)[:N]
  dflow, fc, dc, loss = _tc_final(el, dd2, adj, flow2, dv[:N],
                                  demands[0])

  normalized = norm_f.reshape(NPAD, D)[:N][None]
  pred = pred_f.reshape(NPAD, D)[:N][None]
  return (flow2[None], fc.reshape(1), normalized, dc.reshape(1), pred,
          dflow[None], jnp.zeros((1,), jnp.int32), loss.reshape(1))


# speedup vs baseline: 1.4824x; 1.0072x over previous
import functools

import jax
import jax.numpy as jnp
from jax import lax
from jax.experimental import pallas as pl
from jax.experimental.pallas import tpu as pltpu
from jax.experimental.pallas import tpu_sc as plsc

N = 10000
D = 32
F = 32
EMB = 32
ENC = 64
HID = 32
LAYERS = 2
FLOW_ITERS = 8
DUAL_ITERS = 8
STEP = 0.01
MOM = 0.9
BIG = 1e9

NC = 2
NS = 16
NW = NC * NS
L = 16

NPAD = 10240
NT = NPAD // NW
ET = NT * D
NT0 = 448
NT1 = 192
ET0 = NT0 * D
ET1 = NT1 * D
CH = 256
NCH0 = ET0 // CH
NCH1 = ET1 // CH
CHD = 128
NCHD = ET // CHD
NT_F = NPAD // NS
ET_F = NT_F * D

_MESH = plsc.VectorSubcoreMesh(
    core_axis_name="c", subcore_axis_name="s", num_cores=NC, num_subcores=NS)
_SC_PARAMS = pltpu.CompilerParams(
    needs_layout_passes=False, use_tc_tiling_on_sc=False)


def _enc_body(emb_ref, feat_ref, wenc_ref, benc_ref, wgat_ref, asrc_ref,
              adst_ref, h_ref, ssrc_ref, sdst_ref):
  emb = emb_ref[...]
  nrm = jnp.sqrt(jnp.sum(emb * emb, axis=-1, keepdims=True))
  emb = emb / jnp.maximum(nrm, 1.0)
  x = jnp.concatenate([emb, feat_ref[...]], axis=-1)
  st = jnp.dot(x, wenc_ref[...], preferred_element_type=jnp.float32)
  st = st + benc_ref[...]
  h = jnp.dot(st, wgat_ref[...], preferred_element_type=jnp.float32)
  h_ref[...] = h
  ssrc_ref[...] = jnp.dot(h, asrc_ref[...], preferred_element_type=jnp.float32)
  sdst_ref[...] = jnp.dot(h, adst_ref[...], preferred_element_type=jnp.float32)


def _gat_dense_body(agg_ref, wgat_ref, asrc_ref, adst_ref,
                    h_ref, ssrc_ref, sdst_ref):
  st = jnp.tanh(agg_ref[...])
  h = jnp.dot(st, wgat_ref[...], preferred_element_type=jnp.float32)
  h_ref[...] = h
  ssrc_ref[...] = jnp.dot(h, asrc_ref[...], preferred_element_type=jnp.float32)
  sdst_ref[...] = jnp.dot(h, adst_ref[...], preferred_element_type=jnp.float32)


def _dec_dense_body(agg_ref, w1a_ref, w1b_ref, wdu1_ref, bdu1_ref, wdu2_ref,
                    bdu2_ref, u_ref, w_ref, dv_ref):
  enc = jnp.tanh(agg_ref[...])
  u_ref[...] = jnp.dot(enc, w1a_ref[...], preferred_element_type=jnp.float32)
  w_ref[...] = jnp.dot(enc, w1b_ref[...], preferred_element_type=jnp.float32)
  hdu = jnp.tanh(
      jnp.dot(enc, wdu1_ref[...], preferred_element_type=jnp.float32)
      + bdu1_ref[...])
  dv_ref[...] = (jnp.dot(hdu, wdu2_ref[...], preferred_element_type=jnp.float32)
                 + bdu2_ref[...])


def _final_body(el_ref, dd_ref, adj_ref, flow_ref, dv_ref, dem_ref,
                dflow_ref, fc_ref, dc_ref, loss_ref):
  pid = pl.program_id(0)
  el = el_ref[...]
  dd = dd_ref[...]
  am = 1.0 - (adj_ref[...] == N).astype(jnp.float32)
  x = jnp.zeros_like(el)
  v = jnp.zeros_like(el)
  for _ in range(DUAL_ITERS):
    g = 2.0 * el * x + dd
    v = MOM * v - STEP * g
    x = jnp.maximum(x + v, 0.0) * am
  dflow_ref[...] = x
  fl = flow_ref[...]
  fpart = jnp.sum(el * fl * fl)
  dpart = jnp.sum(el * x * x + dd * x) - jnp.sum(dv_ref[...] * dem_ref[...])

  @pl.when(pid == 0)
  def _():
    fc_ref[0, 0] = 0.0
    dc_ref[0, 0] = 0.0

  fc_ref[0, 0] += fpart
  dc_ref[0, 0] += dpart

  @pl.when(pid == pl.num_programs(0) - 1)
  def _():
    loss_ref[0, 0] = fc_ref[0, 0] - dc_ref[0, 0]


def _dyn_bcast(vec, k):
  idx = jnp.full((L,), k, jnp.int32)
  return lax.gather(
      vec, idx[:, None],
      lax.GatherDimensionNumbers(offset_dims=(), collapsed_slice_dims=(0,),
                                 start_index_map=(0,)),
      slice_sizes=(1,),
      mode=lax.GatherScatterMode.PROMISE_IN_BOUNDS)


def _sc_tanh(x):
  ex = jnp.exp(2.0 * x)
  return (ex - 1.0) / (ex + 1.0)


def _gat_sc_body(h_hbm, ssrc_hbm, sdst_hbm, adj_hbm, agg_hbm,
                 sdst_v, ssrc_v, adj_v, nm_v, alpha_v, agg_v, rows0, rows1,
                 sem0, sem1):
  cidx = lax.axis_index("c")
  sidx = lax.axis_index("s")
  is0 = cidx == 0
  n_t = jnp.where(is0, NT0, NT1)
  nb = jnp.where(is0, sidx * NT0, NT0 * NS + sidx * NT1)
  eb = nb * D
  nch = jnp.where(is0, NCH0, NCH1)
  pltpu.sync_copy(sdst_hbm, sdst_v)

  @pl.when(is0)
  def _():
    pltpu.sync_copy(ssrc_hbm.at[pl.ds(sidx * NT0, NT0)],
                    ssrc_v.at[pl.ds(0, NT0)])
    pltpu.sync_copy(adj_hbm.at[pl.ds(sidx * ET0, ET0)],
                    adj_v.at[pl.ds(0, ET0)])

  @pl.when(cidx == 1)
  def _():
    pltpu.sync_copy(ssrc_hbm.at[pl.ds(NT0 * NS + sidx * NT1, NT1)],
                    ssrc_v.at[pl.ds(0, NT1)])
    pltpu.sync_copy(adj_hbm.at[pl.ds(NT0 * NS * D + sidx * ET1, ET1)],
                    adj_v.at[pl.ds(0, ET1)])

  def prep(g, _):
    idx = adj_v[pl.ds(g * L, L)]
    adj_v[pl.ds(g * L, L)] = jnp.where(idx == N, 0, idx)
    nm_v[pl.ds(g * L, L)] = 1.0 - (idx == N).astype(jnp.float32)
    return 0

  lax.fori_loop(0, n_t * D // L, prep, 0)

  def alpha_node(j, _):
    base = j * D
    es = []
    nms = []
    for g in range(2):
      saf = adj_v[pl.ds(base + g * L, L)]
      nmk = nm_v[pl.ds(base + g * L, L)]
      msk = nmk < 0.5
      sg = plsc.load_gather(sdst_v, [saf])
      src = plsc.load_gather(ssrc_v, [jnp.full((L,), j, jnp.int32)])
      e = src + sg
      e = jnp.where(e >= 0.0, e, 0.2 * e)
      e = jnp.where(msk, -BIG, e)
      es.append(e)
      nms.append(nmk)
    mb = jnp.full((L,), jnp.max(jnp.maximum(es[0], es[1])), jnp.float32)
    p0 = jnp.exp(es[0] - mb) * nms[0]
    p1 = jnp.exp(es[1] - mb) * nms[1]
    sb = jnp.full((L,), jnp.sum(p0 + p1), jnp.float32)
    r = 1.0 / jnp.maximum(sb, 1e-30)
    alpha_v[pl.ds(base, L)] = p0 * r
    alpha_v[pl.ds(base + L, L)] = p1 * r
    return 0

  lax.fori_loop(0, n_t, alpha_node, 0)

  def start(ci, rows, sem):
    pltpu.async_copy(h_hbm.at[adj_v.at[pl.ds(ci * CH, CH)]], rows, sem)

  def wait(ci, rows, sem):
    pltpu.make_async_copy(
        h_hbm.at[adj_v.at[pl.ds(ci * CH, CH)]], rows, sem).wait()

  _PROCESS_ANCHOR = 0

  def process(ci, rows):
    for j in range(CH // D):
      a0 = alpha_v[pl.ds(ci * CH + j * D, L)]
      a1 = alpha_v[pl.ds(ci * CH + j * D + L, L)]
      acc = [jnp.zeros((L,), jnp.float32) for _ in range(ENC // L)]
      for dd_ in range(D):
        e_loc = j * D + dd_
        a = _dyn_bcast(a0 if dd_ < L else a1, dd_ % L)
        for f in range(ENC // L):
          acc[f] = acc[f] + a * rows[e_loc, pl.ds(f * L, L)]
      nl = ci * (CH // D) + j
      for f in range(ENC // L):
        agg_v[pl.ds(nl * ENC + f * L, L)] = acc[f]

  start(0, rows0, sem0)
  start(1, rows1, sem1)
  lax.fori_loop(0, n_t, alpha_node, 0)

  def pipe(t, _):
    c0 = 2 * t
    c1 = 2 * t + 1
    wait(c0, rows0, sem0)
    process(c0, rows0)

    @pl.when(c1 + 1 < nch)
    def _():
      start(c1 + 1, rows0, sem0)

    wait(c1, rows1, sem1)
    process(c1, rows1)

    @pl.when(c1 + 2 < nch)
    def _():
      start(c1 + 2, rows1, sem1)

    return 0

  lax.fori_loop(0, nch // 2, pipe, 0)

  @pl.when(is0)
  def _():
    pltpu.sync_copy(agg_v.at[pl.ds(0, NT0 * ENC)],
                    agg_hbm.at[pl.ds(sidx * NT0 * ENC, NT0 * ENC)])

  @pl.when(cidx == 1)
  def _():
    pltpu.sync_copy(agg_v.at[pl.ds(0, NT1 * ENC)],
                    agg_hbm.at[pl.ds((NT0 * NS + sidx * NT1) * ENC,
                                     NT1 * ENC)])


def _dec_sc_body(u_hbm, w_hbm, dv_hbm, adj_hbm, b1_hbm, w2_hbm, b2_hbm,
                 pred_hbm, norm_hbm, ddf_hbm,
                 u_v, dv_v, adj_v, nm_v, b1_v, w2_v, b2_v,
                 pred_v, norm_v, dd_v, rows0, rows1, sem0, sem1):
  cidx = lax.axis_index("c")
  sidx = lax.axis_index("s")
  wid = sidx * NC + cidx
  nb = wid * NT
  eb = wid * ET
  pltpu.sync_copy(u_hbm.at[pl.ds(nb * HID, NT * HID)], u_v)
  pltpu.sync_copy(dv_hbm, dv_v)
  pltpu.sync_copy(adj_hbm.at[pl.ds(eb, ET)], adj_v)
  pltpu.sync_copy(b1_hbm, b1_v)
  pltpu.sync_copy(w2_hbm, w2_v)
  pltpu.sync_copy(b2_hbm, b2_v)

  def prep(g, _):
    idx = adj_v[pl.ds(g * L, L)]
    msk = idx == N
    adj_v[pl.ds(g * L, L)] = jnp.where(msk, 0, idx)
    nm_v[pl.ds(g * L, L)] = 1.0 - msk.astype(jnp.float32)
    return 0

  lax.fori_loop(0, ET // L, prep, 0)

  iota = lax.iota(jnp.int32, L)

  def start(ci, rows, sem):
    pltpu.async_copy(w_hbm.at[adj_v.at[pl.ds(ci * CHD, CHD)]], rows, sem)

  def wait(ci, rows, sem):
    pltpu.make_async_copy(
        w_hbm.at[adj_v.at[pl.ds(ci * CHD, CHD)]], rows, sem).wait()

  def process(ci, rows):
    for j in range(CHD // D):
      nl = ci * (CHD // D) + j
      pws = []
      for g in range(2):
        ebase = j * D + g * L
        e_glob = ci * CHD + ebase
        nmv = nm_v[pl.ds(e_glob, L)]
        row_idx = iota + ebase
        def kbody(k, nw, row_idx=row_idx, nl=nl, nmv=nmv):
          uk = plsc.load_gather(
              u_v, [jnp.full((L,), nl * HID, jnp.int32) + k])
          wk = plsc.load_gather(rows, [row_idx, jnp.full((L,), k, jnp.int32)])
          b1k = plsc.load_gather(b1_v, [jnp.full((L,), k, jnp.int32)])
          w2k = plsc.load_gather(w2_v, [jnp.full((L,), k, jnp.int32)])
          pre = nmv * (uk + wk) + b1k
          ex = jnp.exp(2.0 * pre)
          t = (ex - 1.0) / (ex + 1.0)
          return nw + w2k * t

        nw = lax.fori_loop(0, HID, kbody, jnp.zeros((L,), jnp.float32))
        nw = nw + plsc.load_gather(b2_v, [jnp.zeros((L,), jnp.int32)])
        pw = nw - BIG * (1.0 - nmv)
        pred_v[pl.ds(e_glob, L)] = pw
        pws.append(pw)
        saf = adj_v[pl.ds(e_glob, L)]
        dvg = plsc.load_gather(dv_v, [saf])
        dvn = plsc.load_gather(dv_v, [jnp.full((L,), nb + nl, jnp.int32)])
        dd_v[pl.ds(e_glob, L)] = nmv * (dvg - dvn)
      mb = jnp.full((L,), jnp.max(jnp.maximum(pws[0], pws[1])), jnp.float32)
      p0 = jnp.exp(pws[0] - mb)
      p1 = jnp.exp(pws[1] - mb)
      r = 1.0 / jnp.full((L,), jnp.sum(p0 + p1), jnp.float32)
      e0 = ci * CHD + j * D
      norm_v[pl.ds(e0, L)] = p0 * r
      norm_v[pl.ds(e0 + L, L)] = p1 * r

  start(0, rows0, sem0)

  def pipe(t, _):
    c0 = 2 * t
    c1 = 2 * t + 1
    start(c1, rows1, sem1)
    wait(c0, rows0, sem0)
    process(c0, rows0)

    @pl.when(c1 + 1 < NCHD)
    def _():
      start(c1 + 1, rows0, sem0)

    wait(c1, rows1, sem1)
    process(c1, rows1)
    return 0

  lax.fori_loop(0, NCHD // 2, pipe, 0)
  pltpu.sync_copy(pred_v, pred_hbm.at[pl.ds(eb, ET)])
  pltpu.sync_copy(norm_v, norm_hbm.at[pl.ds(eb, ET)])
  pltpu.sync_copy(dd_v, ddf_hbm.at[pl.ds(eb, ET)])


def _flow_sc_body(norm_hbm, inidx_hbm, dem_hbm, flow_hbm,
                  flow_sh, norm_v, inidx_v, dem_v, infl_v, flow_v, sem):
  cidx = lax.axis_index("c")
  sidx = lax.axis_index("s")

  @pl.when(cidx == 0)
  def _():
    tb_n = sidx * NT_F
    tb_e = sidx * ET_F
    pltpu.sync_copy(norm_hbm.at[pl.ds(tb_e, ET_F)], norm_v)
    pltpu.sync_copy(inidx_hbm.at[pl.ds(tb_e, ET_F)], inidx_v)
    pltpu.sync_copy(dem_hbm.at[pl.ds(tb_n, NT_F)], dem_v)

    zero16 = jnp.zeros((L,), jnp.float32)

    def zinit(i, _):
      flow_v[pl.ds(i * L, L)] = zero16
      return 0

    lax.fori_loop(0, ET_F // L, zinit, 0)
    pltpu.sync_copy(flow_v, flow_sh.at[pl.ds(tb_e, ET_F)])
    plsc.subcore_barrier()

    def one_iter(it, _):
      pltpu.async_copy(flow_sh.at[inidx_v], infl_v, sem).wait()
      plsc.subcore_barrier()

      def node(j, _):
        base = j * D
        i0 = infl_v[pl.ds(base, L)]
        i1 = infl_v[pl.ds(base + L, L)]
        dem = plsc.load_gather(dem_v, [jnp.full((L,), j, jnp.int32)])
        sb = jnp.full((L,), jnp.sum(i0 + i1), jnp.float32)
        tot = jnp.maximum(dem + sb, 0.0)
        flow_v[pl.ds(base, L)] = norm_v[pl.ds(base, L)] * tot
        flow_v[pl.ds(base + L, L)] = norm_v[pl.ds(base + L, L)] * tot
        return 0

      lax.fori_loop(0, NT_F, node, 0)
      pltpu.sync_copy(flow_v, flow_sh.at[pl.ds(tb_e, ET_F)])
      plsc.subcore_barrier()
      return 0

    lax.fori_loop(0, FLOW_ITERS, one_iter, 0)
    pltpu.sync_copy(flow_v, flow_hbm.at[pl.ds(tb_e, ET_F)])


_BLK = 1024


def _tc_enc(emb_p, feat_p, wenc, benc, wgat, asrc, adst):
  grid = (NPAD // _BLK,)
  full = lambda a: pl.BlockSpec(a.shape, lambda i: (0,) * a.ndim)
  return pl.pallas_call(
      _enc_body,
      grid=grid,
      in_specs=[
          pl.BlockSpec((_BLK, EMB), lambda i: (i, 0)),
          pl.BlockSpec((_BLK, F), lambda i: (i, 0)),
          full(wenc), full(benc), full(wgat), full(asrc), full(adst),
      ],
      out_specs=[
          pl.BlockSpec((_BLK, ENC), lambda i: (i, 0)),
          pl.BlockSpec((_BLK, 1), lambda i: (i, 0)),
          pl.BlockSpec((_BLK, 1), lambda i: (i, 0)),
      ],
      out_shape=[
          jax.ShapeDtypeStruct((NPAD, ENC), jnp.float32),
          jax.ShapeDtypeStruct((NPAD, 1), jnp.float32),
          jax.ShapeDtypeStruct((NPAD, 1), jnp.float32),
      ],
  )(emb_p, feat_p, wenc, benc, wgat, asrc, adst)


def _tc_gat_dense(agg, wgat, asrc, adst):
  grid = (NPAD // _BLK,)
  full = lambda a: pl.BlockSpec(a.shape, lambda i: (0,) * a.ndim)
  return pl.pallas_call(
      _gat_dense_body,
      grid=grid,
      in_specs=[
          pl.BlockSpec((_BLK, ENC), lambda i: (i, 0)),
          full(wgat), full(asrc), full(adst),
      ],
      out_specs=[
          pl.BlockSpec((_BLK, ENC), lambda i: (i, 0)),
          pl.BlockSpec((_BLK, 1), lambda i: (i, 0)),
          pl.BlockSpec((_BLK, 1), lambda i: (i, 0)),
      ],
      out_shape=[
          jax.ShapeDtypeStruct((NPAD, ENC), jnp.float32),
          jax.ShapeDtypeStruct((NPAD, 1), jnp.float32),
          jax.ShapeDtypeStruct((NPAD, 1), jnp.float32),
      ],
  )(agg, wgat, asrc, adst)


def _tc_dec_dense(agg, w1a, w1b, wdu1, bdu1, wdu2, bdu2):
  grid = (NPAD // _BLK,)
  full = lambda a: pl.BlockSpec(a.shape, lambda i: (0,) * a.ndim)
  return pl.pallas_call(
      _dec_dense_body,
      grid=grid,
      in_specs=[
          pl.BlockSpec((_BLK, ENC), lambda i: (i, 0)),
          full(w1a), full(w1b), full(wdu1), full(bdu1), full(wdu2), full(bdu2),
      ],
      out_specs=[
          pl.BlockSpec((_BLK, HID), lambda i: (i, 0)),
          pl.BlockSpec((_BLK, HID), lambda i: (i, 0)),
          pl.BlockSpec((_BLK, 1), lambda i: (i, 0)),
      ],
      out_shape=[
          jax.ShapeDtypeStruct((NPAD, HID), jnp.float32),
          jax.ShapeDtypeStruct((NPAD, HID), jnp.float32),
          jax.ShapeDtypeStruct((NPAD, 1), jnp.float32),
      ],
  )(agg, w1a, w1b, wdu1, bdu1, wdu2, bdu2)


_FBLK = 1000


def _tc_final(el, ddm, adj, flow, dv, dem):
  grid = (N // _FBLK,)
  one = lambda: pl.BlockSpec((1, 1), lambda i: (0, 0),
                             memory_space=pltpu.SMEM)
  return pl.pallas_call(
      _final_body,
      grid=grid,
      in_specs=[
          pl.BlockSpec((_FBLK, D), lambda i: (i, 0)),
          pl.BlockSpec((_FBLK, D), lambda i: (i, 0)),
          pl.BlockSpec((_FBLK, D), lambda i: (i, 0)),
          pl.BlockSpec((_FBLK, D), lambda i: (i, 0)),
          pl.BlockSpec((_FBLK, 1), lambda i: (i, 0)),
          pl.BlockSpec((_FBLK, 1), lambda i: (i, 0)),
      ],
      out_specs=[pl.BlockSpec((_FBLK, D), lambda i: (i, 0)), one(), one(),
                 one()],
      out_shape=[
          jax.ShapeDtypeStruct((N, D), jnp.float32),
          jax.ShapeDtypeStruct((1, 1), jnp.float32),
          jax.ShapeDtypeStruct((1, 1), jnp.float32),
          jax.ShapeDtypeStruct((1, 1), jnp.float32),
      ],
  )(el, ddm, adj, flow, dv, dem)


_gat_sc = functools.partial(
    pl.kernel,
    out_type=[jax.ShapeDtypeStruct((NPAD * ENC,), jnp.float32)],
    mesh=_MESH,
    scratch_types=[
        pltpu.VMEM((NPAD,), jnp.float32),
        pltpu.VMEM((NT0,), jnp.float32),
        pltpu.VMEM((ET0,), jnp.int32),
        pltpu.VMEM((ET0,), jnp.float32),
        pltpu.VMEM((ET0,), jnp.float32),
        pltpu.VMEM((NT0 * ENC,), jnp.float32),
        pltpu.VMEM((CH, ENC), jnp.float32),
        pltpu.VMEM((CH, ENC), jnp.float32),
        pltpu.SemaphoreType.DMA,
        pltpu.SemaphoreType.DMA,
    ],
    compiler_params=_SC_PARAMS,
)(_gat_sc_body)


_dec_sc = functools.partial(
    pl.kernel,
    out_type=[
        jax.ShapeDtypeStruct((NPAD * D,), jnp.float32),
        jax.ShapeDtypeStruct((NPAD * D,), jnp.float32),
        jax.ShapeDtypeStruct((NPAD * D,), jnp.float32),
    ],
    mesh=_MESH,
    scratch_types=[
        pltpu.VMEM((NT * HID,), jnp.float32),
        pltpu.VMEM((NPAD,), jnp.float32),
        pltpu.VMEM((ET,), jnp.int32),
        pltpu.VMEM((ET,), jnp.float32),
        pltpu.VMEM((HID,), jnp.float32),
        pltpu.VMEM((HID,), jnp.float32),
        pltpu.VMEM((16,), jnp.float32),
        pltpu.VMEM((ET,), jnp.float32),
        pltpu.VMEM((ET,), jnp.float32),
        pltpu.VMEM((ET,), jnp.float32),
        pltpu.VMEM((CHD, HID), jnp.float32),
        pltpu.VMEM((CHD, HID), jnp.float32),
        pltpu.SemaphoreType.DMA,
        pltpu.SemaphoreType.DMA,
    ],
    compiler_params=_SC_PARAMS,
)(_dec_sc_body)


_flow_sc = functools.partial(
    pl.kernel,
    out_type=[jax.ShapeDtypeStruct((NPAD * D,), jnp.float32)],
    mesh=_MESH,
    scratch_types=[
        pltpu.VMEM_SHARED((NPAD * D,), jnp.float32),
        pltpu.VMEM((ET_F,), jnp.float32),
        pltpu.VMEM((ET_F,), jnp.int32),
        pltpu.VMEM((NT_F,), jnp.float32),
        pltpu.VMEM((ET_F,), jnp.float32),
        pltpu.VMEM((ET_F,), jnp.float32),
        pltpu.SemaphoreType.DMA,
    ],
    compiler_params=_SC_PARAMS,
)(_flow_sc_body)


def kernel(demands, node_features, adj_lst, inv_adj_lst, edge_lengths,
           norm_edge_lengths, common_neighbors, neighborhoods, in_indices,
           rev_indices, num_nodes, emb_table, W_enc, b_enc, W_gat, a_src,
           a_dst, W_dec1, b_dec1, W_dec2, b_dec2, W_du1, b_du1, W_du2, b_du2):
  del inv_adj_lst, norm_edge_lengths, common_neighbors, neighborhoods
  del rev_indices, num_nodes
  pad_n = NPAD - N

  dem = demands[0, :, 0]
  feat = node_features[0]
  adj = adj_lst[0]
  el = edge_lengths[0]
  in_idx = in_indices[0]

  emb_p = jnp.pad(emb_table, ((0, pad_n), (0, 0)))
  feat_p = jnp.pad(feat, ((0, pad_n), (0, 0)))
  adjf = jnp.pad(adj, ((0, pad_n), (0, 0)), constant_values=N).reshape(-1)
  inf_p = jnp.pad(in_idx, ((0, pad_n), (0, 0))).reshape(-1)
  dem_p = jnp.pad(dem, (0, pad_n))

  benc = b_enc.reshape(1, ENC)
  asrc = a_src.reshape(ENC, 1)
  adst = a_dst.reshape(ENC, 1)
  w1a = W_dec1[:ENC]
  w1b = W_dec1[ENC:]
  bdu1 = b_du1.reshape(1, HID)
  bdu2 = b_du2.reshape(1, 1)
  b2_p = jnp.pad(b_dec2, (0, 15))

  h, ssrc, sdst = _tc_enc(emb_p, feat_p, W_enc, benc, W_gat, asrc, adst)
  (agg1,) = _gat_sc(h, ssrc.reshape(-1), sdst.reshape(-1), adjf)
  h2, ssrc2, sdst2 = _tc_gat_dense(agg1.reshape(NPAD, ENC), W_gat, asrc, adst)
  (agg2,) = _gat_sc(h2, ssrc2.reshape(-1), sdst2.reshape(-1), adjf)
  u, w, dv = _tc_dec_dense(agg2.reshape(NPAD, ENC), w1a, w1b, W_du1, bdu1,
                           W_du2, bdu2)
  pred_f, norm_f, dd_f = _dec_sc(u.reshape(-1), w, dv.reshape(-1), adjf,
                                 b_dec1, W_dec2.reshape(-1), b2_p)
  (flow_f,) = _flow_sc(norm_f, inf_p, dem_p)

  flow2 = flow_f.reshape(NPAD, D)[:N]
  dd2 = dd_f.reshape(NPAD, D)[:N]
  dflow, fc, dc, loss = _tc_final(el, dd2, adj, flow2, dv[:N],
                                  demands[0])

  normalized = norm_f.reshape(NPAD, D)[:N][None]
  pred = pred_f.reshape(NPAD, D)[:N][None]
  return (flow2[None], fc.reshape(1), normalized, dc.reshape(1), pred,
          dflow[None], jnp.zeros((1,), jnp.int32), loss.reshape(1))


# speedup vs baseline: 1.4866x; 1.0028x over previous
import functools

import jax
import jax.numpy as jnp
from jax import lax
from jax.experimental import pallas as pl
from jax.experimental.pallas import tpu as pltpu
from jax.experimental.pallas import tpu_sc as plsc

N = 10000
D = 32
F = 32
EMB = 32
ENC = 64
HID = 32
LAYERS = 2
FLOW_ITERS = 8
DUAL_ITERS = 8
STEP = 0.01
MOM = 0.9
BIG = 1e9

NC = 2
NS = 16
NW = NC * NS
L = 16

NPAD = 10240
NT = NPAD // NW
ET = NT * D
NT0 = 448
NT1 = 192
ET0 = NT0 * D
ET1 = NT1 * D
CH = 256
NCH0 = ET0 // CH
NCH1 = ET1 // CH
CHD = 128
NCHD = ET // CHD
NT_F = NPAD // NS
ET_F = NT_F * D

_MESH = plsc.VectorSubcoreMesh(
    core_axis_name="c", subcore_axis_name="s", num_cores=NC, num_subcores=NS)
_SC_PARAMS = pltpu.CompilerParams(
    needs_layout_passes=False, use_tc_tiling_on_sc=False)


def _enc_body(emb_ref, feat_ref, wenc_ref, benc_ref, wgat_ref, asrc_ref,
              adst_ref, h_ref, ssrc_ref, sdst_ref):
  emb = emb_ref[...]
  nrm = jnp.sqrt(jnp.sum(emb * emb, axis=-1, keepdims=True))
  emb = emb / jnp.maximum(nrm, 1.0)
  x = jnp.concatenate([emb, feat_ref[...]], axis=-1)
  st = jnp.dot(x, wenc_ref[...], preferred_element_type=jnp.float32)
  st = st + benc_ref[...]
  h = jnp.dot(st, wgat_ref[...], preferred_element_type=jnp.float32)
  h_ref[...] = h
  ssrc_ref[...] = jnp.dot(h, asrc_ref[...], preferred_element_type=jnp.float32)
  sdst_ref[...] = jnp.dot(h, adst_ref[...], preferred_element_type=jnp.float32)


def _gat_dense_body(agg_ref, wgat_ref, asrc_ref, adst_ref,
                    h_ref, ssrc_ref, sdst_ref):
  st = jnp.tanh(agg_ref[...])
  h = jnp.dot(st, wgat_ref[...], preferred_element_type=jnp.float32)
  h_ref[...] = h
  ssrc_ref[...] = jnp.dot(h, asrc_ref[...], preferred_element_type=jnp.float32)
  sdst_ref[...] = jnp.dot(h, adst_ref[...], preferred_element_type=jnp.float32)


def _dec_dense_body(agg_ref, w1a_ref, w1b_ref, wdu1_ref, bdu1_ref, wdu2_ref,
                    bdu2_ref, u_ref, w_ref, dv_ref):
  enc = jnp.tanh(agg_ref[...])
  u_ref[...] = jnp.dot(enc, w1a_ref[...], preferred_element_type=jnp.float32)
  w_ref[...] = jnp.dot(enc, w1b_ref[...], preferred_element_type=jnp.float32)
  hdu = jnp.tanh(
      jnp.dot(enc, wdu1_ref[...], preferred_element_type=jnp.float32)
      + bdu1_ref[...])
  dv_ref[...] = (jnp.dot(hdu, wdu2_ref[...], preferred_element_type=jnp.float32)
                 + bdu2_ref[...])


def _final_body(el_ref, dd_ref, adj_ref, flow_ref, dv_ref, dem_ref,
                dflow_ref, fc_ref, dc_ref, loss_ref):
  pid = pl.program_id(0)
  el = el_ref[...]
  dd = dd_ref[...]
  am = 1.0 - (adj_ref[...] == N).astype(jnp.float32)
  x = jnp.zeros_like(el)
  v = jnp.zeros_like(el)
  for _ in range(DUAL_ITERS):
    g = 2.0 * el * x + dd
    v = MOM * v - STEP * g
    x = jnp.maximum(x + v, 0.0) * am
  dflow_ref[...] = x
  fl = flow_ref[...]
  fpart = jnp.sum(el * fl * fl)
  dpart = jnp.sum(el * x * x + dd * x) - jnp.sum(dv_ref[...] * dem_ref[...])

  @pl.when(pid == 0)
  def _():
    fc_ref[0, 0] = 0.0
    dc_ref[0, 0] = 0.0

  fc_ref[0, 0] += fpart
  dc_ref[0, 0] += dpart

  @pl.when(pid == pl.num_programs(0) - 1)
  def _():
    loss_ref[0, 0] = fc_ref[0, 0] - dc_ref[0, 0]


def _dyn_bcast(vec, k):
  idx = jnp.full((L,), k, jnp.int32)
  return lax.gather(
      vec, idx[:, None],
      lax.GatherDimensionNumbers(offset_dims=(), collapsed_slice_dims=(0,),
                                 start_index_map=(0,)),
      slice_sizes=(1,),
      mode=lax.GatherScatterMode.PROMISE_IN_BOUNDS)


def _sc_tanh(x):
  ex = jnp.exp(2.0 * x)
  return (ex - 1.0) / (ex + 1.0)


def _gat_sc_body(h_hbm, ssrc_hbm, sdst_hbm, adj_hbm, agg_hbm,
                 sdst_v, ssrc_v, adj_v, alpha_v, agg_v, rows0, rows1,
                 sem0, sem1):
  cidx = lax.axis_index("c")
  sidx = lax.axis_index("s")
  is0 = cidx == 0
  n_t = jnp.where(is0, NT0, NT1)
  nb = jnp.where(is0, sidx * NT0, NT0 * NS + sidx * NT1)
  eb = nb * D
  nch = jnp.where(is0, NCH0, NCH1)
  pltpu.sync_copy(sdst_hbm, sdst_v)

  @pl.when(is0)
  def _():
    pltpu.sync_copy(ssrc_hbm.at[pl.ds(sidx * NT0, NT0)],
                    ssrc_v.at[pl.ds(0, NT0)])
    pltpu.sync_copy(adj_hbm.at[pl.ds(sidx * ET0, ET0)],
                    adj_v.at[pl.ds(0, ET0)])

  @pl.when(cidx == 1)
  def _():
    pltpu.sync_copy(ssrc_hbm.at[pl.ds(NT0 * NS + sidx * NT1, NT1)],
                    ssrc_v.at[pl.ds(0, NT1)])
    pltpu.sync_copy(adj_hbm.at[pl.ds(NT0 * NS * D + sidx * ET1, ET1)],
                    adj_v.at[pl.ds(0, ET1)])

  def alpha_node(j, _):
    base = j * D
    es = []
    nms = []
    for g in range(2):
      idx = adj_v[pl.ds(base + g * L, L)]
      msk = idx == N
      adj_v[pl.ds(base + g * L, L)] = jnp.where(msk, 0, idx)
      sg = plsc.load_gather(sdst_v, [jnp.where(msk, 0, idx)])
      src = plsc.load_gather(ssrc_v, [jnp.full((L,), j, jnp.int32)])
      e = src + sg
      e = jnp.where(e >= 0.0, e, 0.2 * e)
      e = jnp.where(msk, -BIG, e)
      es.append(e)
      nms.append(1.0 - msk.astype(jnp.float32))
    mb = jnp.full((L,), jnp.max(jnp.maximum(es[0], es[1])), jnp.float32)
    p0 = jnp.exp(es[0] - mb) * nms[0]
    p1 = jnp.exp(es[1] - mb) * nms[1]
    sb = jnp.full((L,), jnp.sum(p0 + p1), jnp.float32)
    r = 1.0 / jnp.maximum(sb, 1e-30)
    alpha_v[pl.ds(base, L)] = p0 * r
    alpha_v[pl.ds(base + L, L)] = p1 * r
    return 0

  lax.fori_loop(0, n_t, alpha_node, 0)

  def start(ci, rows, sem):
    pltpu.async_copy(h_hbm.at[adj_v.at[pl.ds(ci * CH, CH)]], rows, sem)

  def wait(ci, rows, sem):
    pltpu.make_async_copy(
        h_hbm.at[adj_v.at[pl.ds(ci * CH, CH)]], rows, sem).wait()

  def process(ci, rows):
    for j in range(CH // D):
      a0 = alpha_v[pl.ds(ci * CH + j * D, L)]
      a1 = alpha_v[pl.ds(ci * CH + j * D + L, L)]
      acc = [jnp.zeros((L,), jnp.float32) for _ in range(ENC // L)]
      for dd_ in range(D):
        e_loc = j * D + dd_
        a = _dyn_bcast(a0 if dd_ < L else a1, dd_ % L)
        for f in range(ENC // L):
          acc[f] = acc[f] + a * rows[e_loc, pl.ds(f * L, L)]
      nl = ci * (CH // D) + j
      for f in range(ENC // L):
        agg_v[pl.ds(nl * ENC + f * L, L)] = acc[f]

  start(0, rows0, sem0)

  def pipe(t, _):
    c0 = 2 * t
    c1 = 2 * t + 1
    start(c1, rows1, sem1)
    wait(c0, rows0, sem0)
    process(c0, rows0)

    @pl.when(c1 + 1 < nch)
    def _():
      start(c1 + 1, rows0, sem0)

    wait(c1, rows1, sem1)
    process(c1, rows1)
    return 0

  lax.fori_loop(0, nch // 2, pipe, 0)

  @pl.when(is0)
  def _():
    pltpu.sync_copy(agg_v.at[pl.ds(0, NT0 * ENC)],
                    agg_hbm.at[pl.ds(sidx * NT0 * ENC, NT0 * ENC)])

  @pl.when(cidx == 1)
  def _():
    pltpu.sync_copy(agg_v.at[pl.ds(0, NT1 * ENC)],
                    agg_hbm.at[pl.ds((NT0 * NS + sidx * NT1) * ENC,
                                     NT1 * ENC)])


def _dec_sc_body(u_hbm, w_hbm, dv_hbm, adj_hbm, b1_hbm, w2_hbm, b2_hbm,
                 pred_hbm, norm_hbm, ddf_hbm,
                 u_v, dv_v, adj_v, nm_v, b1_v, w2_v, b2_v,
                 pred_v, norm_v, dd_v, rows0, rows1, sem0, sem1):
  cidx = lax.axis_index("c")
  sidx = lax.axis_index("s")
  wid = sidx * NC + cidx
  nb = wid * NT
  eb = wid * ET
  pltpu.sync_copy(u_hbm.at[pl.ds(nb * HID, NT * HID)], u_v)
  pltpu.sync_copy(dv_hbm, dv_v)
  pltpu.sync_copy(adj_hbm.at[pl.ds(eb, ET)], adj_v)
  pltpu.sync_copy(b1_hbm, b1_v)
  pltpu.sync_copy(w2_hbm, w2_v)
  pltpu.sync_copy(b2_hbm, b2_v)

  def prep(g, _):
    idx = adj_v[pl.ds(g * L, L)]
    msk = idx == N
    adj_v[pl.ds(g * L, L)] = jnp.where(msk, 0, idx)
    nm_v[pl.ds(g * L, L)] = 1.0 - msk.astype(jnp.float32)
    return 0

  lax.fori_loop(0, ET // L, prep, 0)

  iota = lax.iota(jnp.int32, L)

  def start(ci, rows, sem):
    pltpu.async_copy(w_hbm.at[adj_v.at[pl.ds(ci * CHD, CHD)]], rows, sem)

  def wait(ci, rows, sem):
    pltpu.make_async_copy(
        w_hbm.at[adj_v.at[pl.ds(ci * CHD, CHD)]], rows, sem).wait()

  def process(ci, rows):
    for j in range(CHD // D):
      nl = ci * (CHD // D) + j
      pws = []
      for g in range(2):
        ebase = j * D + g * L
        e_glob = ci * CHD + ebase
        nmv = nm_v[pl.ds(e_glob, L)]
        row_idx = iota + ebase

        def kbody(k, nw, row_idx=row_idx, nl=nl, nmv=nmv):
          uk = plsc.load_gather(
              u_v, [jnp.full((L,), nl * HID, jnp.int32) + k])
          wk = plsc.load_gather(rows, [row_idx, jnp.full((L,), k, jnp.int32)])
          b1k = plsc.load_gather(b1_v, [jnp.full((L,), k, jnp.int32)])
          w2k = plsc.load_gather(w2_v, [jnp.full((L,), k, jnp.int32)])
          pre = nmv * (uk + wk) + b1k
          ex = jnp.exp(2.0 * pre)
          t = (ex - 1.0) / (ex + 1.0)
          return nw + w2k * t

        nw = lax.fori_loop(0, HID, kbody, jnp.zeros((L,), jnp.float32))
        nw = nw + plsc.load_gather(b2_v, [jnp.zeros((L,), jnp.int32)])
        pw = nw - BIG * (1.0 - nmv)
        pred_v[pl.ds(e_glob, L)] = pw
        pws.append(pw)
        saf = adj_v[pl.ds(e_glob, L)]
        dvg = plsc.load_gather(dv_v, [saf])
        dvn = plsc.load_gather(dv_v, [jnp.full((L,), nb + nl, jnp.int32)])
        dd_v[pl.ds(e_glob, L)] = nmv * (dvg - dvn)
      mb = jnp.full((L,), jnp.max(jnp.maximum(pws[0], pws[1])), jnp.float32)
      p0 = jnp.exp(pws[0] - mb)
      p1 = jnp.exp(pws[1] - mb)
      r = 1.0 / jnp.full((L,), jnp.sum(p0 + p1), jnp.float32)
      e0 = ci * CHD + j * D
      norm_v[pl.ds(e0, L)] = p0 * r
      norm_v[pl.ds(e0 + L, L)] = p1 * r

  start(0, rows0, sem0)

  def pipe(t, _):
    c0 = 2 * t
    c1 = 2 * t + 1
    start(c1, rows1, sem1)
    wait(c0, rows0, sem0)
    process(c0, rows0)

    @pl.when(c1 + 1 < NCHD)
    def _():
      start(c1 + 1, rows0, sem0)

    wait(c1, rows1, sem1)
    process(c1, rows1)
    return 0

  lax.fori_loop(0, NCHD // 2, pipe, 0)
  pltpu.sync_copy(pred_v, pred_hbm.at[pl.ds(eb, ET)])
  pltpu.sync_copy(norm_v, norm_hbm.at[pl.ds(eb, ET)])
  pltpu.sync_copy(dd_v, ddf_hbm.at[pl.ds(eb, ET)])


def _flow_sc_body(norm_hbm, inidx_hbm, dem_hbm, flow_hbm,
                  flow_sh, norm_v, inidx_v, dem_v, infl_v, flow_v, sem):
  cidx = lax.axis_index("c")
  sidx = lax.axis_index("s")

  @pl.when(cidx == 0)
  def _():
    tb_n = sidx * NT_F
    tb_e = sidx * ET_F
    pltpu.sync_copy(norm_hbm.at[pl.ds(tb_e, ET_F)], norm_v)
    pltpu.sync_copy(inidx_hbm.at[pl.ds(tb_e, ET_F)], inidx_v)
    pltpu.sync_copy(dem_hbm.at[pl.ds(tb_n, NT_F)], dem_v)

    zero16 = jnp.zeros((L,), jnp.float32)

    def zinit(i, _):
      flow_v[pl.ds(i * L, L)] = zero16
      return 0

    lax.fori_loop(0, ET_F // L, zinit, 0)
    pltpu.sync_copy(flow_v, flow_sh.at[pl.ds(tb_e, ET_F)])
    plsc.subcore_barrier()

    def one_iter(it, _):
      pltpu.async_copy(flow_sh.at[inidx_v], infl_v, sem).wait()
      plsc.subcore_barrier()

      def node(j, _):
        base = j * D
        i0 = infl_v[pl.ds(base, L)]
        i1 = infl_v[pl.ds(base + L, L)]
        dem = plsc.load_gather(dem_v, [jnp.full((L,), j, jnp.int32)])
        sb = jnp.full((L,), jnp.sum(i0 + i1), jnp.float32)
        tot = jnp.maximum(dem + sb, 0.0)
        flow_v[pl.ds(base, L)] = norm_v[pl.ds(base, L)] * tot
        flow_v[pl.ds(base + L, L)] = norm_v[pl.ds(base + L, L)] * tot
        return 0

      lax.fori_loop(0, NT_F, node, 0)
      pltpu.sync_copy(flow_v, flow_sh.at[pl.ds(tb_e, ET_F)])
      plsc.subcore_barrier()
      return 0

    lax.fori_loop(0, FLOW_ITERS, one_iter, 0)
    pltpu.sync_copy(flow_v, flow_hbm.at[pl.ds(tb_e, ET_F)])


_BLK = 1024


def _tc_enc(emb_p, feat_p, wenc, benc, wgat, asrc, adst):
  grid = (NPAD // _BLK,)
  full = lambda a: pl.BlockSpec(a.shape, lambda i: (0,) * a.ndim)
  return pl.pallas_call(
      _enc_body,
      grid=grid,
      in_specs=[
          pl.BlockSpec((_BLK, EMB), lambda i: (i, 0)),
          pl.BlockSpec((_BLK, F), lambda i: (i, 0)),
          full(wenc), full(benc), full(wgat), full(asrc), full(adst),
      ],
      out_specs=[
          pl.BlockSpec((_BLK, ENC), lambda i: (i, 0)),
          pl.BlockSpec((_BLK, 1), lambda i: (i, 0)),
          pl.BlockSpec((_BLK, 1), lambda i: (i, 0)),
      ],
      out_shape=[
          jax.ShapeDtypeStruct((NPAD, ENC), jnp.float32),
          jax.ShapeDtypeStruct((NPAD, 1), jnp.float32),
          jax.ShapeDtypeStruct((NPAD, 1), jnp.float32),
      ],
  )(emb_p, feat_p, wenc, benc, wgat, asrc, adst)


def _tc_gat_dense(agg, wgat, asrc, adst):
  grid = (NPAD // _BLK,)
  full = lambda a: pl.BlockSpec(a.shape, lambda i: (0,) * a.ndim)
  return pl.pallas_call(
      _gat_dense_body,
      grid=grid,
      in_specs=[
          pl.BlockSpec((_BLK, ENC), lambda i: (i, 0)),
          full(wgat), full(asrc), full(adst),
      ],
      out_specs=[
          pl.BlockSpec((_BLK, ENC), lambda i: (i, 0)),
          pl.BlockSpec((_BLK, 1), lambda i: (i, 0)),
          pl.BlockSpec((_BLK, 1), lambda i: (i, 0)),
      ],
      out_shape=[
          jax.ShapeDtypeStruct((NPAD, ENC), jnp.float32),
          jax.ShapeDtypeStruct((NPAD, 1), jnp.float32),
          jax.ShapeDtypeStruct((NPAD, 1), jnp.float32),
      ],
  )(agg, wgat, asrc, adst)


def _tc_dec_dense(agg, w1a, w1b, wdu1, bdu1, wdu2, bdu2):
  grid = (NPAD // _BLK,)
  full = lambda a: pl.BlockSpec(a.shape, lambda i: (0,) * a.ndim)
  return pl.pallas_call(
      _dec_dense_body,
      grid=grid,
      in_specs=[
          pl.BlockSpec((_BLK, ENC), lambda i: (i, 0)),
          full(w1a), full(w1b), full(wdu1), full(bdu1), full(wdu2), full(bdu2),
      ],
      out_specs=[
          pl.BlockSpec((_BLK, HID), lambda i: (i, 0)),
          pl.BlockSpec((_BLK, HID), lambda i: (i, 0)),
          pl.BlockSpec((_BLK, 1), lambda i: (i, 0)),
      ],
      out_shape=[
          jax.ShapeDtypeStruct((NPAD, HID), jnp.float32),
          jax.ShapeDtypeStruct((NPAD, HID), jnp.float32),
          jax.ShapeDtypeStruct((NPAD, 1), jnp.float32),
      ],
  )(agg, w1a, w1b, wdu1, bdu1, wdu2, bdu2)


_FBLK = 1000


def _tc_final(el, ddm, adj, flow, dv, dem):
  grid = (N // _FBLK,)
  one = lambda: pl.BlockSpec((1, 1), lambda i: (0, 0),
                             memory_space=pltpu.SMEM)
  return pl.pallas_call(
      _final_body,
      grid=grid,
      in_specs=[
          pl.BlockSpec((_FBLK, D), lambda i: (i, 0)),
          pl.BlockSpec((_FBLK, D), lambda i: (i, 0)),
          pl.BlockSpec((_FBLK, D), lambda i: (i, 0)),
          pl.BlockSpec((_FBLK, D), lambda i: (i, 0)),
          pl.BlockSpec((_FBLK, 1), lambda i: (i, 0)),
          pl.BlockSpec((_FBLK, 1), lambda i: (i, 0)),
      ],
      out_specs=[pl.BlockSpec((_FBLK, D), lambda i: (i, 0)), one(), one(),
                 one()],
      out_shape=[
          jax.ShapeDtypeStruct((N, D), jnp.float32),
          jax.ShapeDtypeStruct((1, 1), jnp.float32),
          jax.ShapeDtypeStruct((1, 1), jnp.float32),
          jax.ShapeDtypeStruct((1, 1), jnp.float32),
      ],
  )(el, ddm, adj, flow, dv, dem)


_gat_sc = functools.partial(
    pl.kernel,
    out_type=[jax.ShapeDtypeStruct((NPAD * ENC,), jnp.float32)],
    mesh=_MESH,
    scratch_types=[
        pltpu.VMEM((NPAD,), jnp.float32),
        pltpu.VMEM((NT0,), jnp.float32),
        pltpu.VMEM((ET0,), jnp.int32),
        pltpu.VMEM((ET0,), jnp.float32),
        pltpu.VMEM((NT0 * ENC,), jnp.float32),
        pltpu.VMEM((CH, ENC), jnp.float32),
        pltpu.VMEM((CH, ENC), jnp.float32),
        pltpu.SemaphoreType.DMA,
        pltpu.SemaphoreType.DMA,
    ],
    compiler_params=_SC_PARAMS,
)(_gat_sc_body)


_dec_sc = functools.partial(
    pl.kernel,
    out_type=[
        jax.ShapeDtypeStruct((NPAD * D,), jnp.float32),
        jax.ShapeDtypeStruct((NPAD * D,), jnp.float32),
        jax.ShapeDtypeStruct((NPAD * D,), jnp.float32),
    ],
    mesh=_MESH,
    scratch_types=[
        pltpu.VMEM((NT * HID,), jnp.float32),
        pltpu.VMEM((NPAD,), jnp.float32),
        pltpu.VMEM((ET,), jnp.int32),
        pltpu.VMEM((ET,), jnp.float32),
        pltpu.VMEM((HID,), jnp.float32),
        pltpu.VMEM((HID,), jnp.float32),
        pltpu.VMEM((16,), jnp.float32),
        pltpu.VMEM((ET,), jnp.float32),
        pltpu.VMEM((ET,), jnp.float32),
        pltpu.VMEM((ET,), jnp.float32),
        pltpu.VMEM((CHD, HID), jnp.float32),
        pltpu.VMEM((CHD, HID), jnp.float32),
        pltpu.SemaphoreType.DMA,
        pltpu.SemaphoreType.DMA,
    ],
    compiler_params=_SC_PARAMS,
)(_dec_sc_body)


_flow_sc = functools.partial(
    pl.kernel,
    out_type=[jax.ShapeDtypeStruct((NPAD * D,), jnp.float32)],
    mesh=_MESH,
    scratch_types=[
        pltpu.VMEM_SHARED((NPAD * D,), jnp.float32),
        pltpu.VMEM((ET_F,), jnp.float32),
        pltpu.VMEM((ET_F,), jnp.int32),
        pltpu.VMEM((NT_F,), jnp.float32),
        pltpu.VMEM((ET_F,), jnp.float32),
        pltpu.VMEM((ET_F,), jnp.float32),
        pltpu.SemaphoreType.DMA,
    ],
    compiler_params=_SC_PARAMS,
)(_flow_sc_body)


def kernel(demands, node_features, adj_lst, inv_adj_lst, edge_lengths,
           norm_edge_lengths, common_neighbors, neighborhoods, in_indices,
           rev_indices, num_nodes, emb_table, W_enc, b_enc, W_gat, a_src,
           a_dst, W_dec1, b_dec1, W_dec2, b_dec2, W_du1, b_du1, W_du2, b_du2):
  del inv_adj_lst, norm_edge_lengths, common_neighbors, neighborhoods
  del rev_indices, num_nodes
  pad_n = NPAD - N

  dem = demands[0, :, 0]
  feat = node_features[0]
  adj = adj_lst[0]
  el = edge_lengths[0]
  in_idx = in_indices[0]

  emb_p = jnp.pad(emb_table, ((0, pad_n), (0, 0)))
  feat_p = jnp.pad(feat, ((0, pad_n), (0, 0)))
  adjf = jnp.pad(adj, ((0, pad_n), (0, 0)), constant_values=N).reshape(-1)
  inf_p = jnp.pad(in_idx, ((0, pad_n), (0, 0))).reshape(-1)
  dem_p = jnp.pad(dem, (0, pad_n))

  benc = b_enc.reshape(1, ENC)
  asrc = a_src.reshape(ENC, 1)
  adst = a_dst.reshape(ENC, 1)
  w1a = W_dec1[:ENC]
  w1b = W_dec1[ENC:]
  bdu1 = b_du1.reshape(1, HID)
  bdu2 = b_du2.reshape(1, 1)
  b2_p = jnp.pad(b_dec2, (0, 15))

  h, ssrc, sdst = _tc_enc(emb_p, feat_p, W_enc, benc, W_gat, asrc, adst)
  (agg1,) = _gat_sc(h, ssrc.reshape(-1), sdst.reshape(-1), adjf)
  h2, ssrc2, sdst2 = _tc_gat_dense(agg1.reshape(NPAD, ENC), W_gat, asrc, adst)
  (agg2,) = _gat_sc(h2, ssrc2.reshape(-1), sdst2.reshape(-1), adjf)
  u, w, dv = _tc_dec_dense(agg2.reshape(NPAD, ENC), w1a, w1b, W_du1, bdu1,
                           W_du2, bdu2)
  pred_f, norm_f, dd_f = _dec_sc(u.reshape(-1), w, dv.reshape(-1), adjf,
                                 b_dec1, W_dec2.reshape(-1), b2_p)
  (flow_f,) = _flow_sc(norm_f, inf_p, dem_p)

  flow2 = flow_f.reshape(NPAD, D)[:N]
  dd2 = dd_f.reshape(NPAD, D)[:N]
  dflow, fc, dc, loss = _tc_final(el, dd2, adj, flow2, dv[:N],
                                  demands[0])

  normalized = norm_f.reshape(NPAD, D)[:N][None]
  pred = pred_f.reshape(NPAD, D)[:N][None]
  return (flow2[None], fc.reshape(1), normalized, dc.reshape(1), pred,
          dflow[None], jnp.zeros((1,), jnp.int32), loss.reshape(1))
